# Initial kernel scaffold; baseline (speedup 1.0000x reference)
import jax
import jax.numpy as jnp
from jax.experimental import pallas as pl


def kernel(seq, edge_index, W_fc, b_fc, W_gat, att_src, att_dst, bias_gat, prelu_a):
    raise NotImplementedError("write your pallas kernel here")


# speedup vs baseline: 7.3071x; 7.3071x over previous
import jax
import jax.numpy as jnp
from jax import lax
from jax.experimental import pallas as pl
from jax.experimental.pallas import tpu as pltpu
from jax.experimental.pallas import tpu_sc as plsc

N = 10000
E = 160000
FT_IN = 512
NB = 256
NBH = NB // 2
NC, NS, LANES = 2, 16, 16
NW = NC * NS
EPW = 5120
EP = NW * EPW
CH = 64
NCHUNK = EPW // CH
ROWS = 10240
RPS = ROWS // NS
RB = 1000
GRID = N // RB


def _mm_body(seq_ref, wfc_ref, bfc_ref, wgat_ref, asv_ref, adv_ref,
             h0_ref, h1_ref, as_ref, ad_ref, mx_ref):
    x = seq_ref[...]
    ret = lax.dot_general(x, wfc_ref[...], (((1,), (1,)), ((), ())),
                          precision=lax.Precision.HIGHEST) + bfc_ref[...]
    h = lax.dot_general(ret, wgat_ref[...], (((1,), (1,)), ((), ())),
                        precision=lax.Precision.HIGHEST)
    a_s = jnp.sum(h * asv_ref[...], axis=1)
    a_d = jnp.sum(h * adv_ref[...], axis=1)
    h0_ref[...] = h[:, :NBH]
    h1_ref[...] = h[:, NBH:]
    as_ref[...] = a_s[:, None]
    ad_ref[...] = a_d[:, None]
    bm = jnp.stack([jnp.max(a_s), jnp.max(a_d)])[None, :]

    @pl.when(pl.program_id(0) == 0)
    def _():
        mx_ref[...] = bm

    @pl.when(pl.program_id(0) > 0)
    def _():
        mx_ref[...] = jnp.maximum(mx_ref[...], bm)


def _sc_edge_kernel(src_hbm, dst_hbm, asrc_hbm, adst_hbm, m_hbm, h0_hbm,
                    h1_hbm, part_hbm, wpart_hbm,
                    src_v, dst_v, atab, m_v, w_v, rowbuf, wstage,
                    feat_acc, w_acc, sem):
    cid = lax.axis_index("c")
    sid = lax.axis_index("s")
    wid = sid * NC + cid
    row0 = wid * NCHUNK

    pltpu.sync_copy(src_hbm.at[pl.ds(row0, NCHUNK)], src_v)
    pltpu.sync_copy(dst_hbm.at[pl.ds(row0, NCHUNK)], dst_v)
    pltpu.sync_copy(m_hbm, m_v)

    m16 = m_v[...]
    iota = lax.iota(jnp.int32, LANES)

    pltpu.sync_copy(asrc_hbm, atab)

    @pl.loop(0, NCHUNK)
    def _(j):
        @pl.loop(0, CH // LANES)
        def _(k):
            s16 = src_v[j, pl.ds(k * LANES, LANES)]
            w_v[pl.ds(j * CH + k * LANES, LANES)] = plsc.load_gather(
                atab, [s16])

    pltpu.sync_copy(adst_hbm, atab)

    @pl.loop(0, NCHUNK)
    def _(j):
        @pl.loop(0, CH // LANES)
        def _(k):
            d16 = dst_v[j, pl.ds(k * LANES, LANES)]
            u = w_v[pl.ds(j * CH + k * LANES, LANES)] + plsc.load_gather(
                atab, [d16])
            e = jnp.where(u >= 0.0, u, 0.2 * u)
            w_v[pl.ds(j * CH + k * LANES, LANES)] = jnp.exp(e - m16)

    for f in range(2):
        h_hbm = h0_hbm if f == 0 else h1_hbm

        @pl.loop(0, CH)
        def _(i):
            for c in range(NBH // LANES):
                rowbuf[i, pl.ds(c * LANES, LANES)] = jnp.zeros(
                    (LANES,), jnp.float32)
            if f == 0:
                wstage[i, pl.ds(0, LANES)] = jnp.zeros((LANES,), jnp.float32)

        @pl.loop(0, RPS // CH)
        def _(r):
            off = sid * RPS + r * CH
            pltpu.sync_copy(rowbuf, feat_acc.at[pl.ds(off, CH)])
            if f == 0:
                pltpu.sync_copy(wstage, w_acc.at[pl.ds(off, CH)])

        plsc.subcore_barrier()

        @pl.loop(0, NCHUNK)
        def _(j):
            pltpu.async_copy(h_hbm.at[src_v.at[j]], rowbuf, sem).wait()

            @pl.loop(0, CH)
            def _(i):
                idx16 = iota * 0 + (j * CH + i)
                wb = plsc.load_gather(w_v, [idx16])
                for c in range(NBH // LANES):
                    rowbuf[i, pl.ds(c * LANES, LANES)] = (
                        rowbuf[i, pl.ds(c * LANES, LANES)] * wb)
                if f == 0:
                    wstage[i, pl.ds(0, LANES)] = jnp.where(iota == 0, wb, 0.0)

            pltpu.sync_copy(rowbuf, feat_acc.at[dst_v.at[j]], add=True)
            if f == 0:
                pltpu.sync_copy(wstage, w_acc.at[dst_v.at[j]], add=True)

        plsc.subcore_barrier()

        @pl.loop(0, RPS // CH)
        def _(r):
            off = sid * RPS + r * CH
            pltpu.sync_copy(feat_acc.at[pl.ds(off, CH)],
                            part_hbm.at[cid, f, pl.ds(off, CH)])
            if f == 0:
                pltpu.sync_copy(w_acc.at[pl.ds(off, CH)],
                                wpart_hbm.at[cid, pl.ds(off, CH)])

        plsc.subcore_barrier()


def _combine_body(part_ref, wpart_ref, h0_ref, h1_ref, as_ref, ad_ref, m_ref,
                  b_ref, p_ref, out_ref):
    P = part_ref[...]
    W = wpart_ref[...]
    num0 = P[0, 0] + P[1, 0]
    num1 = P[0, 1] + P[1, 1]
    wsum = W[0, :, 0] + W[1, :, 0]
    M = m_ref[0, 0] + m_ref[0, 1]
    u = as_ref[...][:, 0] + ad_ref[...][:, 0]
    e = jnp.where(u >= 0.0, u, 0.2 * u)
    wself = jnp.exp(e - M)
    denom = wsum + wself
    o0 = (num0 + wself[:, None] * h0_ref[...]) / denom[:, None]
    o1 = (num1 + wself[:, None] * h1_ref[...]) / denom[:, None]
    o = jnp.concatenate([o0, o1], axis=1) + b_ref[...]
    a = p_ref[0, 0]
    out_ref[...] = jnp.where(o >= 0.0, o, a * o)


def kernel(seq, edge_index, W_fc, b_fc, W_gat, att_src, att_dst, bias_gat,
           prelu_a):
    f32 = jnp.float32

    h0, h1, as2, ad2, mx = pl.pallas_call(
        _mm_body,
        grid=(GRID,),
        in_specs=[
            pl.BlockSpec((RB, FT_IN), lambda i: (i, 0)),
            pl.BlockSpec((NB, FT_IN), lambda i: (0, 0)),
            pl.BlockSpec((1, NB), lambda i: (0, 0)),
            pl.BlockSpec((NB, NB), lambda i: (0, 0)),
            pl.BlockSpec((1, NB), lambda i: (0, 0)),
            pl.BlockSpec((1, NB), lambda i: (0, 0)),
        ],
        out_specs=[
            pl.BlockSpec((RB, NBH), lambda i: (i, 0)),
            pl.BlockSpec((RB, NBH), lambda i: (i, 0)),
            pl.BlockSpec((RB, 1), lambda i: (i, 0)),
            pl.BlockSpec((RB, 1), lambda i: (i, 0)),
            pl.BlockSpec((1, 2), lambda i: (0, 0)),
        ],
        out_shape=[
            jax.ShapeDtypeStruct((N, NBH), f32),
            jax.ShapeDtypeStruct((N, NBH), f32),
            jax.ShapeDtypeStruct((N, 1), f32),
            jax.ShapeDtypeStruct((N, 1), f32),
            jax.ShapeDtypeStruct((1, 2), f32),
        ],
    )(seq, W_fc, b_fc.reshape(1, NB), W_gat, att_src.reshape(1, NB),
      att_dst.reshape(1, NB))

    src = jnp.concatenate(
        [edge_index[0], jnp.zeros((EP - E,), jnp.int32)]).reshape(
            NW * NCHUNK, CH)
    dst = jnp.concatenate(
        [edge_index[1], jnp.full((EP - E,), N, jnp.int32)]).reshape(
            NW * NCHUNK, CH)
    asrc_p = jnp.pad(as2[:, 0], (0, ROWS - N))
    adst_p = jnp.pad(ad2[:, 0], (0, ROWS - N))
    m16 = jnp.full((LANES,), mx[0, 0] + mx[0, 1], f32)

    mesh = plsc.VectorSubcoreMesh(core_axis_name="c", subcore_axis_name="s")
    cp = pltpu.CompilerParams(needs_layout_passes=False,
                              use_tc_tiling_on_sc=False)
    sc_kernel = pl.kernel(
        _sc_edge_kernel,
        out_type=[
            jax.ShapeDtypeStruct((NC, 2, ROWS, NBH), f32),
            jax.ShapeDtypeStruct((NC, ROWS, LANES), f32),
        ],
        mesh=mesh,
        compiler_params=cp,
        scratch_types=[
            pltpu.VMEM((NCHUNK, CH), jnp.int32),
            pltpu.VMEM((NCHUNK, CH), jnp.int32),
            pltpu.VMEM((ROWS,), f32),
            pltpu.VMEM((LANES,), f32),
            pltpu.VMEM((EPW,), f32),
            pltpu.VMEM((CH, NBH), f32),
            pltpu.VMEM((CH, LANES), f32),
            pltpu.VMEM_SHARED((ROWS, NBH), f32),
            pltpu.VMEM_SHARED((ROWS, LANES), f32),
            pltpu.SemaphoreType.DMA,
        ],
    )
    part, wpart = sc_kernel(src, dst, asrc_p, adst_p, m16, h0, h1)

    out = pl.pallas_call(
        _combine_body,
        grid=(GRID,),
        in_specs=[
            pl.BlockSpec((NC, 2, RB, NBH), lambda i: (0, 0, i, 0)),
            pl.BlockSpec((NC, RB, LANES), lambda i: (0, i, 0)),
            pl.BlockSpec((RB, NBH), lambda i: (i, 0)),
            pl.BlockSpec((RB, NBH), lambda i: (i, 0)),
            pl.BlockSpec((RB, 1), lambda i: (i, 0)),
            pl.BlockSpec((RB, 1), lambda i: (i, 0)),
            pl.BlockSpec((1, 2), lambda i: (0, 0)),
            pl.BlockSpec((1, NB), lambda i: (0, 0)),
            pl.BlockSpec((1, 1), lambda i: (0, 0)),
        ],
        out_specs=pl.BlockSpec((RB, NB), lambda i: (i, 0)),
        out_shape=jax.ShapeDtypeStruct((N, NB), f32),
    )(part, wpart, h0, h1, as2, ad2, mx, bias_gat.reshape(1, NB),
      prelu_a.reshape(1, 1))
    return out


# speedup vs baseline: 8.0932x; 1.1076x over previous
import jax
import jax.numpy as jnp
from jax import lax
from jax.experimental import pallas as pl
from jax.experimental.pallas import tpu as pltpu
from jax.experimental.pallas import tpu_sc as plsc

N = 10000
E = 160000
FT_IN = 512
NB = 256
NBH = NB // 2
NC, NS, LANES = 2, 16, 16
NW = NC * NS
EPW = 5120
EP = NW * EPW
CH = 64
NCHUNK = EPW // CH
ROWS = 10240
RPS = ROWS // NS
RB = 1000
GRID = N // RB


def _mm_body(seq_ref, wfc_ref, bfc_ref, wgat_ref, asv_ref, adv_ref,
             h0_ref, h1_ref, as_ref, ad_ref, mx_ref):
    x = seq_ref[...]
    ret = lax.dot_general(x, wfc_ref[...], (((1,), (1,)), ((), ())),
                          precision=lax.Precision.HIGHEST) + bfc_ref[...]
    h = lax.dot_general(ret, wgat_ref[...], (((1,), (1,)), ((), ())),
                        precision=lax.Precision.HIGHEST)
    a_s = jnp.sum(h * asv_ref[...], axis=1)
    a_d = jnp.sum(h * adv_ref[...], axis=1)
    h0_ref[...] = h[:, :NBH]
    h1_ref[...] = h[:, NBH:]
    as_ref[...] = a_s[:, None]
    ad_ref[...] = a_d[:, None]
    bm = jnp.stack([jnp.max(a_s), jnp.max(a_d)])[None, :]

    @pl.when(pl.program_id(0) == 0)
    def _():
        mx_ref[...] = bm

    @pl.when(pl.program_id(0) > 0)
    def _():
        mx_ref[...] = jnp.maximum(mx_ref[...], bm)


def _sc_edge_kernel(src_hbm, dst_hbm, asrc_hbm, adst_hbm, m_hbm, h0_hbm,
                    h1_hbm, part_hbm, wpart_hbm,
                    src_v, dst_v, m_v, w_v,
                    feat_acc, w_acc, gsem0, gsem1):
    cid = lax.axis_index("c")
    sid = lax.axis_index("s")
    wid = sid * NC + cid
    row0 = wid * NCHUNK

    pltpu.sync_copy(src_hbm.at[pl.ds(row0, NCHUNK)],
                    src_v.at[pl.ds(0, NCHUNK)])
    pltpu.sync_copy(dst_hbm.at[pl.ds(row0, NCHUNK)], dst_v)
    pltpu.sync_copy(m_hbm, m_v)
    for k in range(CH // LANES):
        src_v[NCHUNK, pl.ds(k * LANES, LANES)] = jnp.zeros((LANES,), jnp.int32)

    m16 = m_v[...]
    iota = lax.iota(jnp.int32, LANES)

    def _weights(atab):
        pltpu.sync_copy(asrc_hbm, atab)

        @pl.loop(0, NCHUNK)
        def _(j):
            @pl.loop(0, CH // LANES)
            def _(k):
                s16 = src_v[j, pl.ds(k * LANES, LANES)]
                w_v[pl.ds(j * CH + k * LANES, LANES)] = plsc.load_gather(
                    atab, [s16])

        pltpu.sync_copy(adst_hbm, atab)

        @pl.loop(0, NCHUNK)
        def _(j):
            @pl.loop(0, CH // LANES)
            def _(k):
                d16 = dst_v[j, pl.ds(k * LANES, LANES)]
                u = w_v[pl.ds(j * CH + k * LANES, LANES)] + plsc.load_gather(
                    atab, [d16])
                e = jnp.where(u >= 0.0, u, 0.2 * u)
                w_v[pl.ds(j * CH + k * LANES, LANES)] = jnp.exp(e - m16)

    pl.run_scoped(_weights, pltpu.VMEM((ROWS,), jnp.float32))

    def _passes(rb0, rb1, wstage):
        _feature_passes(src_v, dst_v, w_v, h0_hbm, h1_hbm, part_hbm,
                        wpart_hbm, feat_acc, w_acc, gsem0, gsem1, cid, sid,
                        iota, rb0, rb1, wstage)

    pl.run_scoped(_passes,
                  pltpu.VMEM((CH, NBH), jnp.float32),
                  pltpu.VMEM((CH, NBH), jnp.float32),
                  pltpu.VMEM((CH, LANES), jnp.float32))


def _feature_passes(src_v, dst_v, w_v, h0_hbm, h1_hbm, part_hbm, wpart_hbm,
                    feat_acc, w_acc, gsem0, gsem1, cid, sid, iota,
                    rb0, rb1, wstage):
    def _scale(rb, j, f):
        @pl.loop(0, CH)
        def _(i):
            idx16 = iota * 0 + (j * CH + i)
            wb = plsc.load_gather(w_v, [idx16])
            for c in range(NBH // LANES):
                rb[i, pl.ds(c * LANES, LANES)] = (
                    rb[i, pl.ds(c * LANES, LANES)] * wb)
            if f == 0:
                wstage[i, pl.ds(0, LANES)] = jnp.where(iota == 0, wb, 0.0)

    for f in range(2):
        h_hbm = h0_hbm if f == 0 else h1_hbm

        @pl.loop(0, CH)
        def _(i):
            for c in range(NBH // LANES):
                rb0[i, pl.ds(c * LANES, LANES)] = jnp.zeros(
                    (LANES,), jnp.float32)
            if f == 0:
                wstage[i, pl.ds(0, LANES)] = jnp.zeros((LANES,), jnp.float32)

        @pl.loop(0, RPS // CH)
        def _(r):
            off = sid * RPS + r * CH
            pltpu.sync_copy(rb0, feat_acc.at[pl.ds(off, CH)])
            if f == 0:
                pltpu.sync_copy(wstage, w_acc.at[pl.ds(off, CH)])

        plsc.subcore_barrier()

        pltpu.make_async_copy(h_hbm.at[src_v.at[0]], rb0, gsem0).start()

        @pl.loop(0, NCHUNK // 2)
        def _(t):
            c0 = 2 * t
            c1 = c0 + 1
            pltpu.make_async_copy(h_hbm.at[src_v.at[c1]], rb1, gsem1).start()
            pltpu.make_async_copy(h_hbm.at[src_v.at[c0]], rb0, gsem0).wait()
            _scale(rb0, c0, f)
            pltpu.sync_copy(rb0, feat_acc.at[dst_v.at[c0]], add=True)
            if f == 0:
                pltpu.sync_copy(wstage, w_acc.at[dst_v.at[c0]], add=True)
            pltpu.make_async_copy(h_hbm.at[src_v.at[c0 + 2]], rb0,
                                  gsem0).start()
            pltpu.make_async_copy(h_hbm.at[src_v.at[c1]], rb1, gsem1).wait()
            _scale(rb1, c1, f)
            pltpu.sync_copy(rb1, feat_acc.at[dst_v.at[c1]], add=True)
            if f == 0:
                pltpu.sync_copy(wstage, w_acc.at[dst_v.at[c1]], add=True)

        pltpu.make_async_copy(h_hbm.at[src_v.at[NCHUNK]], rb0, gsem0).wait()

        plsc.subcore_barrier()

        @pl.loop(0, RPS // CH)
        def _(r):
            off = sid * RPS + r * CH
            pltpu.sync_copy(feat_acc.at[pl.ds(off, CH)],
                            part_hbm.at[cid, f, pl.ds(off, CH)])
            if f == 0:
                pltpu.sync_copy(w_acc.at[pl.ds(off, CH)],
                                wpart_hbm.at[cid, pl.ds(off, CH)])

        plsc.subcore_barrier()


def _combine_body(part_ref, wpart_ref, h0_ref, h1_ref, as_ref, ad_ref, m_ref,
                  b_ref, p_ref, out_ref):
    P = part_ref[...]
    W = wpart_ref[...]
    num0 = P[0, 0] + P[1, 0]
    num1 = P[0, 1] + P[1, 1]
    wsum = W[0, :, 0] + W[1, :, 0]
    M = m_ref[0, 0] + m_ref[0, 1]
    u = as_ref[...][:, 0] + ad_ref[...][:, 0]
    e = jnp.where(u >= 0.0, u, 0.2 * u)
    wself = jnp.exp(e - M)
    denom = wsum + wself
    o0 = (num0 + wself[:, None] * h0_ref[...]) / denom[:, None]
    o1 = (num1 + wself[:, None] * h1_ref[...]) / denom[:, None]
    o = jnp.concatenate([o0, o1], axis=1) + b_ref[...]
    a = p_ref[0, 0]
    out_ref[...] = jnp.where(o >= 0.0, o, a * o)


def kernel(seq, edge_index, W_fc, b_fc, W_gat, att_src, att_dst, bias_gat,
           prelu_a):
    f32 = jnp.float32

    h0, h1, as2, ad2, mx = pl.pallas_call(
        _mm_body,
        grid=(GRID,),
        in_specs=[
            pl.BlockSpec((RB, FT_IN), lambda i: (i, 0)),
            pl.BlockSpec((NB, FT_IN), lambda i: (0, 0)),
            pl.BlockSpec((1, NB), lambda i: (0, 0)),
            pl.BlockSpec((NB, NB), lambda i: (0, 0)),
            pl.BlockSpec((1, NB), lambda i: (0, 0)),
            pl.BlockSpec((1, NB), lambda i: (0, 0)),
        ],
        out_specs=[
            pl.BlockSpec((RB, NBH), lambda i: (i, 0)),
            pl.BlockSpec((RB, NBH), lambda i: (i, 0)),
            pl.BlockSpec((RB, 1), lambda i: (i, 0)),
            pl.BlockSpec((RB, 1), lambda i: (i, 0)),
            pl.BlockSpec((1, 2), lambda i: (0, 0)),
        ],
        out_shape=[
            jax.ShapeDtypeStruct((N, NBH), f32),
            jax.ShapeDtypeStruct((N, NBH), f32),
            jax.ShapeDtypeStruct((N, 1), f32),
            jax.ShapeDtypeStruct((N, 1), f32),
            jax.ShapeDtypeStruct((1, 2), f32),
        ],
    )(seq, W_fc, b_fc.reshape(1, NB), W_gat, att_src.reshape(1, NB),
      att_dst.reshape(1, NB))

    src = jnp.concatenate(
        [edge_index[0], jnp.zeros((EP - E,), jnp.int32)]).reshape(
            NW * NCHUNK, CH)
    dst = jnp.concatenate(
        [edge_index[1], jnp.full((EP - E,), N, jnp.int32)]).reshape(
            NW * NCHUNK, CH)
    asrc_p = jnp.pad(as2[:, 0], (0, ROWS - N))
    adst_p = jnp.pad(ad2[:, 0], (0, ROWS - N))
    m16 = jnp.full((LANES,), mx[0, 0] + mx[0, 1], f32)

    mesh = plsc.VectorSubcoreMesh(core_axis_name="c", subcore_axis_name="s")
    cp = pltpu.CompilerParams(needs_layout_passes=False,
                              use_tc_tiling_on_sc=False)
    sc_kernel = pl.kernel(
        _sc_edge_kernel,
        out_type=[
            jax.ShapeDtypeStruct((NC, 2, ROWS, NBH), f32),
            jax.ShapeDtypeStruct((NC, ROWS, LANES), f32),
        ],
        mesh=mesh,
        compiler_params=cp,
        scratch_types=[
            pltpu.VMEM((NCHUNK + 1, CH), jnp.int32),
            pltpu.VMEM((NCHUNK, CH), jnp.int32),
            pltpu.VMEM((LANES,), f32),
            pltpu.VMEM((EPW,), f32),
            pltpu.VMEM_SHARED((ROWS, NBH), f32),
            pltpu.VMEM_SHARED((ROWS, LANES), f32),
            pltpu.SemaphoreType.DMA,
            pltpu.SemaphoreType.DMA,
        ],
    )
    part, wpart = sc_kernel(src, dst, asrc_p, adst_p, m16, h0, h1)

    out = pl.pallas_call(
        _combine_body,
        grid=(GRID,),
        in_specs=[
            pl.BlockSpec((NC, 2, RB, NBH), lambda i: (0, 0, i, 0)),
            pl.BlockSpec((NC, RB, LANES), lambda i: (0, i, 0)),
            pl.BlockSpec((RB, NBH), lambda i: (i, 0)),
            pl.BlockSpec((RB, NBH), lambda i: (i, 0)),
            pl.BlockSpec((RB, 1), lambda i: (i, 0)),
            pl.BlockSpec((RB, 1), lambda i: (i, 0)),
            pl.BlockSpec((1, 2), lambda i: (0, 0)),
            pl.BlockSpec((1, NB), lambda i: (0, 0)),
            pl.BlockSpec((1, 1), lambda i: (0, 0)),
        ],
        out_specs=pl.BlockSpec((RB, NB), lambda i: (i, 0)),
        out_shape=jax.ShapeDtypeStruct((N, NB), f32),
    )(part, wpart, h0, h1, as2, ad2, mx, bias_gat.reshape(1, NB),
      prelu_a.reshape(1, 1))
    return out


# speedup vs baseline: 10.1924x; 1.2594x over previous
import jax
import jax.numpy as jnp
from jax import lax
from jax.experimental import pallas as pl
from jax.experimental.pallas import tpu as pltpu
from jax.experimental.pallas import tpu_sc as plsc

N = 10000
E = 160000
FT_IN = 512
NB = 256
NBH = NB // 2
NC, NS, LANES = 2, 16, 16
NW = NC * NS
CH = 64
NCH0 = 100
NCH1 = 60
NCHROWS = NS * (NCH0 + NCH1)
EP = NCHROWS * CH
NCHPAD = NCHROWS + NCH0 + 1 - NCH1
EPPAD = 2640 * CH
ROWS = 10240
RPS = ROWS // NS
RB = 1000
GRID = N // RB


def _mm_body(seq_ref, wfc_ref, bfc_ref, wgat_ref, asv_ref, adv_ref,
             h0_ref, h1_ref, as_ref, ad_ref, mx_ref):
    x = seq_ref[...]
    ret = lax.dot_general(x, wfc_ref[...], (((1,), (1,)), ((), ())),
                          precision=lax.Precision.HIGHEST) + bfc_ref[...]
    h = lax.dot_general(ret, wgat_ref[...], (((1,), (1,)), ((), ())),
                        precision=lax.Precision.HIGHEST)
    a_s = jnp.sum(h * asv_ref[...], axis=1)
    a_d = jnp.sum(h * adv_ref[...], axis=1)
    h0_ref[...] = h[:, :NBH]
    h1_ref[...] = h[:, NBH:]
    as_ref[...] = a_s[:, None]
    ad_ref[...] = a_d[:, None]
    bm = jnp.stack([jnp.max(a_s), jnp.max(a_d)])[None, :]

    @pl.when(pl.program_id(0) == 0)
    def _():
        mx_ref[...] = bm

    @pl.when(pl.program_id(0) > 0)
    def _():
        mx_ref[...] = jnp.maximum(mx_ref[...], bm)


def _sc_edge_kernel(src_hbm, dst_hbm, asrc_hbm, adst_hbm, m_hbm, h0_hbm,
                    h1_hbm, part_hbm, wpart_hbm,
                    src_v, dst_v, m_v, w_v,
                    feat_acc, w_acc, gsem0, gsem1):
    cid = lax.axis_index("c")
    sid = lax.axis_index("s")
    nchunk = jnp.where(cid == 0, NCH0, NCH1)
    row0 = jnp.where(cid == 0, sid * NCH0, NS * NCH0 + sid * NCH1)

    pltpu.sync_copy(src_hbm.at[pl.ds(row0, NCH0 + 1)], src_v)
    pltpu.sync_copy(dst_hbm.at[pl.ds(row0, NCH0)], dst_v)
    pltpu.sync_copy(m_hbm, m_v)

    m16 = m_v[...]
    iota = lax.iota(jnp.int32, LANES)

    def _weights(atab):
        pltpu.sync_copy(asrc_hbm, atab)

        @pl.loop(0, nchunk)
        def _(j):
            @pl.loop(0, CH // LANES)
            def _(k):
                s16 = src_v[j, pl.ds(k * LANES, LANES)]
                w_v[pl.ds(j * CH + k * LANES, LANES)] = plsc.load_gather(
                    atab, [s16])

        pltpu.sync_copy(adst_hbm, atab)

        @pl.loop(0, nchunk)
        def _(j):
            @pl.loop(0, CH // LANES)
            def _(k):
                d16 = dst_v[j, pl.ds(k * LANES, LANES)]
                u = w_v[pl.ds(j * CH + k * LANES, LANES)] + plsc.load_gather(
                    atab, [d16])
                e = jnp.where(u >= 0.0, u, 0.2 * u)
                w_v[pl.ds(j * CH + k * LANES, LANES)] = jnp.exp(e - m16)

    pl.run_scoped(_weights, pltpu.VMEM((ROWS,), jnp.float32))

    def _passes(rb0, rb1, wstage):
        _feature_passes(src_v, dst_v, w_v, h0_hbm, h1_hbm, part_hbm,
                        wpart_hbm, feat_acc, w_acc, gsem0, gsem1, cid, sid,
                        iota, nchunk, rb0, rb1, wstage)

    pl.run_scoped(_passes,
                  pltpu.VMEM((CH, NBH), jnp.float32),
                  pltpu.VMEM((CH, NBH), jnp.float32),
                  pltpu.VMEM((CH, LANES), jnp.float32))


def _feature_passes(src_v, dst_v, w_v, h0_hbm, h1_hbm, part_hbm, wpart_hbm,
                    feat_acc, w_acc, gsem0, gsem1, cid, sid, iota, nchunk,
                    rb0, rb1, wstage):
    def _scale(rb, j, f):
        @pl.loop(0, CH)
        def _(i):
            idx16 = iota * 0 + (j * CH + i)
            wb = plsc.load_gather(w_v, [idx16])
            for c in range(NBH // LANES):
                rb[i, pl.ds(c * LANES, LANES)] = (
                    rb[i, pl.ds(c * LANES, LANES)] * wb)
            if f == 0:
                wstage[i, pl.ds(0, LANES)] = jnp.where(iota == 0, wb, 0.0)

    for f in range(2):
        h_hbm = h0_hbm if f == 0 else h1_hbm

        @pl.loop(0, CH)
        def _(i):
            for c in range(NBH // LANES):
                rb0[i, pl.ds(c * LANES, LANES)] = jnp.zeros(
                    (LANES,), jnp.float32)
            if f == 0:
                wstage[i, pl.ds(0, LANES)] = jnp.zeros((LANES,), jnp.float32)

        @pl.loop(0, RPS // CH)
        def _(r):
            off = sid * RPS + r * CH
            pltpu.sync_copy(rb0, feat_acc.at[pl.ds(off, CH)])
            if f == 0:
                pltpu.sync_copy(wstage, w_acc.at[pl.ds(off, CH)])

        plsc.subcore_barrier()

        pltpu.make_async_copy(h_hbm.at[src_v.at[0]], rb0, gsem0).start()

        @pl.loop(0, nchunk // 2)
        def _(t):
            c0 = 2 * t
            c1 = c0 + 1
            pltpu.make_async_copy(h_hbm.at[src_v.at[c1]], rb1, gsem1).start()
            pltpu.make_async_copy(h_hbm.at[src_v.at[c0]], rb0, gsem0).wait()
            _scale(rb0, c0, f)
            pltpu.sync_copy(rb0, feat_acc.at[dst_v.at[c0]], add=True)
            if f == 0:
                pltpu.sync_copy(wstage, w_acc.at[dst_v.at[c0]], add=True)
            pltpu.make_async_copy(h_hbm.at[src_v.at[c0 + 2]], rb0,
                                  gsem0).start()
            pltpu.make_async_copy(h_hbm.at[src_v.at[c1]], rb1, gsem1).wait()
            _scale(rb1, c1, f)
            pltpu.sync_copy(rb1, feat_acc.at[dst_v.at[c1]], add=True)
            if f == 0:
                pltpu.sync_copy(wstage, w_acc.at[dst_v.at[c1]], add=True)

        pltpu.make_async_copy(h_hbm.at[src_v.at[nchunk]], rb0, gsem0).wait()

        plsc.subcore_barrier()

        @pl.loop(0, RPS // CH)
        def _(r):
            off = sid * RPS + r * CH
            pltpu.sync_copy(feat_acc.at[pl.ds(off, CH)],
                            part_hbm.at[cid, f, pl.ds(off, CH)])
            if f == 0:
                pltpu.sync_copy(w_acc.at[pl.ds(off, CH)],
                                wpart_hbm.at[cid, pl.ds(off, CH)])

        plsc.subcore_barrier()


def _combine_body(part_ref, wpart_ref, h0_ref, h1_ref, as_ref, ad_ref, m_ref,
                  b_ref, p_ref, out_ref):
    P = part_ref[...]
    W = wpart_ref[...]
    num0 = P[0, 0] + P[1, 0]
    num1 = P[0, 1] + P[1, 1]
    wsum = W[0, :, 0] + W[1, :, 0]
    M = m_ref[0, 0] + m_ref[0, 1]
    u = as_ref[...][:, 0] + ad_ref[...][:, 0]
    e = jnp.where(u >= 0.0, u, 0.2 * u)
    wself = jnp.exp(e - M)
    denom = wsum + wself
    o0 = (num0 + wself[:, None] * h0_ref[...]) / denom[:, None]
    o1 = (num1 + wself[:, None] * h1_ref[...]) / denom[:, None]
    o = jnp.concatenate([o0, o1], axis=1) + b_ref[...]
    a = p_ref[0, 0]
    out_ref[...] = jnp.where(o >= 0.0, o, a * o)


def kernel(seq, edge_index, W_fc, b_fc, W_gat, att_src, att_dst, bias_gat,
           prelu_a):
    f32 = jnp.float32

    h0, h1, as2, ad2, mx = pl.pallas_call(
        _mm_body,
        grid=(GRID,),
        in_specs=[
            pl.BlockSpec((RB, FT_IN), lambda i: (i, 0)),
            pl.BlockSpec((NB, FT_IN), lambda i: (0, 0)),
            pl.BlockSpec((1, NB), lambda i: (0, 0)),
            pl.BlockSpec((NB, NB), lambda i: (0, 0)),
            pl.BlockSpec((1, NB), lambda i: (0, 0)),
            pl.BlockSpec((1, NB), lambda i: (0, 0)),
        ],
        out_specs=[
            pl.BlockSpec((RB, NBH), lambda i: (i, 0)),
            pl.BlockSpec((RB, NBH), lambda i: (i, 0)),
            pl.BlockSpec((RB, 1), lambda i: (i, 0)),
            pl.BlockSpec((RB, 1), lambda i: (i, 0)),
            pl.BlockSpec((1, 2), lambda i: (0, 0)),
        ],
        out_shape=[
            jax.ShapeDtypeStruct((N, NBH), f32),
            jax.ShapeDtypeStruct((N, NBH), f32),
            jax.ShapeDtypeStruct((N, 1), f32),
            jax.ShapeDtypeStruct((N, 1), f32),
            jax.ShapeDtypeStruct((1, 2), f32),
        ],
    )(seq, W_fc, b_fc.reshape(1, NB), W_gat, att_src.reshape(1, NB),
      att_dst.reshape(1, NB))

    src = jnp.concatenate(
        [edge_index[0], jnp.zeros((EPPAD - E,), jnp.int32)]).reshape(
            EPPAD // CH, CH)
    dst = jnp.concatenate(
        [edge_index[1], jnp.full((EPPAD - E,), N, jnp.int32)]).reshape(
            EPPAD // CH, CH)
    asrc_p = jnp.pad(as2[:, 0], (0, ROWS - N))
    adst_p = jnp.pad(ad2[:, 0], (0, ROWS - N))
    m16 = jnp.full((LANES,), mx[0, 0] + mx[0, 1], f32)

    mesh = plsc.VectorSubcoreMesh(core_axis_name="c", subcore_axis_name="s")
    cp = pltpu.CompilerParams(needs_layout_passes=False,
                              use_tc_tiling_on_sc=False)
    sc_kernel = pl.kernel(
        _sc_edge_kernel,
        out_type=[
            jax.ShapeDtypeStruct((NC, 2, ROWS, NBH), f32),
            jax.ShapeDtypeStruct((NC, ROWS, LANES), f32),
        ],
        mesh=mesh,
        compiler_params=cp,
        scratch_types=[
            pltpu.VMEM((NCH0 + 1, CH), jnp.int32),
            pltpu.VMEM((NCH0, CH), jnp.int32),
            pltpu.VMEM((LANES,), f32),
            pltpu.VMEM((NCH0 * CH,), f32),
            pltpu.VMEM_SHARED((ROWS, NBH), f32),
            pltpu.VMEM_SHARED((ROWS, LANES), f32),
            pltpu.SemaphoreType.DMA,
            pltpu.SemaphoreType.DMA,
        ],
    )
    part, wpart = sc_kernel(src, dst, asrc_p, adst_p, m16, h0, h1)

    out = pl.pallas_call(
        _combine_body,
        grid=(GRID,),
        in_specs=[
            pl.BlockSpec((NC, 2, RB, NBH), lambda i: (0, 0, i, 0)),
            pl.BlockSpec((NC, RB, LANES), lambda i: (0, i, 0)),
            pl.BlockSpec((RB, NBH), lambda i: (i, 0)),
            pl.BlockSpec((RB, NBH), lambda i: (i, 0)),
            pl.BlockSpec((RB, 1), lambda i: (i, 0)),
            pl.BlockSpec((RB, 1), lambda i: (i, 0)),
            pl.BlockSpec((1, 2), lambda i: (0, 0)),
            pl.BlockSpec((1, NB), lambda i: (0, 0)),
            pl.BlockSpec((1, 1), lambda i: (0, 0)),
        ],
        out_specs=pl.BlockSpec((RB, NB), lambda i: (i, 0)),
        out_shape=jax.ShapeDtypeStruct((N, NB), f32),
    )(part, wpart, h0, h1, as2, ad2, mx, bias_gat.reshape(1, NB),
      prelu_a.reshape(1, 1))
    return out


# speedup vs baseline: 13.5079x; 1.3253x over previous
import jax
import jax.numpy as jnp
from jax import lax
from jax.experimental import pallas as pl
from jax.experimental.pallas import tpu as pltpu
from jax.experimental.pallas import tpu_sc as plsc

N = 10000
E = 160000
FT_IN = 512
NB = 256
NBH = NB // 2
NC, NS, LANES = 2, 16, 16
NW = NC * NS
CH = 64
NCH0 = 100
NCH1 = 60
NCHROWS = NS * (NCH0 + NCH1)
EP = NCHROWS * CH
NCHPAD = NCHROWS + NCH0 + 1 - NCH1
EPPAD = 2640 * CH
ROWS = 10240
RPS = ROWS // NS
RB = 1000
GRID = N // RB


def _mm_body(seq_ref, wfc_ref, bfc_ref, wgat_ref, asv_ref, adv_ref,
             h0_ref, h1_ref, as_ref, ad_ref, mx_ref):
    x = seq_ref[...]
    ret = lax.dot_general(x, wfc_ref[...], (((1,), (1,)), ((), ())),
                          precision=lax.Precision.HIGHEST) + bfc_ref[...]
    h = lax.dot_general(ret, wgat_ref[...], (((1,), (1,)), ((), ())),
                        precision=lax.Precision.HIGHEST)
    a_s = jnp.sum(h * asv_ref[...], axis=1)
    a_d = jnp.sum(h * adv_ref[...], axis=1)
    h0_ref[...] = h[:, :NBH]
    h1_ref[...] = h[:, NBH:]
    as_ref[...] = a_s[:, None]
    ad_ref[...] = a_d[:, None]
    bm = jnp.stack([jnp.max(a_s), jnp.max(a_d)])[None, :]

    @pl.when(pl.program_id(0) == 0)
    def _():
        mx_ref[...] = bm

    @pl.when(pl.program_id(0) > 0)
    def _():
        mx_ref[...] = jnp.maximum(mx_ref[...], bm)


def _sc_edge_kernel(src_hbm, dst_hbm, asrc_hbm, adst_hbm, m_hbm, h0_hbm,
                    h1_hbm, part_hbm, wpart_hbm,
                    src_v, dst_v, m_v, w_v,
                    feat_acc, w_acc, gsem0, gsem1):
    cid = lax.axis_index("c")
    sid = lax.axis_index("s")
    nchunk = jnp.where(cid == 0, NCH0, NCH1)
    row0 = jnp.where(cid == 0, sid * NCH0, NS * NCH0 + sid * NCH1)

    pltpu.sync_copy(src_hbm.at[pl.ds(row0, NCH0 + 1)], src_v)
    pltpu.sync_copy(dst_hbm.at[pl.ds(row0, NCH0)], dst_v)
    pltpu.sync_copy(m_hbm, m_v)

    m16 = m_v[...]
    iota = lax.iota(jnp.int32, LANES)

    def _weights(atab):
        pltpu.sync_copy(asrc_hbm, atab)

        @pl.loop(0, nchunk)
        def _(j):
            @pl.loop(0, CH // LANES)
            def _(k):
                s16 = src_v[j, pl.ds(k * LANES, LANES)]
                w_v[pl.ds(j * CH + k * LANES, LANES)] = plsc.load_gather(
                    atab, [s16])

        pltpu.sync_copy(adst_hbm, atab)

        @pl.loop(0, nchunk)
        def _(j):
            @pl.loop(0, CH // LANES)
            def _(k):
                d16 = dst_v[j, pl.ds(k * LANES, LANES)]
                u = w_v[pl.ds(j * CH + k * LANES, LANES)] + plsc.load_gather(
                    atab, [d16])
                e = jnp.where(u >= 0.0, u, 0.2 * u)
                w_v[pl.ds(j * CH + k * LANES, LANES)] = jnp.exp(e - m16)

    pl.run_scoped(_weights, pltpu.VMEM((ROWS,), jnp.float32))

    def _passes(rb0, rb1, stage, wstage):
        _feature_passes(src_v, dst_v, w_v, h0_hbm, h1_hbm, part_hbm,
                        wpart_hbm, feat_acc, w_acc, gsem0, gsem1, cid, sid,
                        iota, nchunk, rb0, rb1, stage, wstage)

    pl.run_scoped(_passes,
                  pltpu.VMEM((CH, NBH), jnp.bfloat16),
                  pltpu.VMEM((CH, NBH), jnp.bfloat16),
                  pltpu.VMEM((CH, NBH), jnp.float32),
                  pltpu.VMEM((CH, LANES), jnp.float32))


def _feature_passes(src_v, dst_v, w_v, h0_hbm, h1_hbm, part_hbm, wpart_hbm,
                    feat_acc, w_acc, gsem0, gsem1, cid, sid, iota, nchunk,
                    rb0, rb1, stage, wstage):
    def _scale(rb, j, f):
        @pl.loop(0, CH)
        def _(i):
            idx16 = iota * 0 + (j * CH + i)
            wb = plsc.load_gather(w_v, [idx16])
            for c in range(NBH // (2 * LANES)):
                pair = rb[i, pl.ds(c * 2 * LANES, 2 * LANES)]
                vi = plsc.bitcast(pair, jnp.int32)
                lo = plsc.bitcast(vi << 16, jnp.float32)
                hi = plsc.bitcast(vi & jnp.int32(-65536), jnp.float32)
                stage[i, pl.ds(c * LANES, LANES)] = lo * wb
                stage[i, pl.ds(NBH // 2 + c * LANES, LANES)] = hi * wb
            if f == 0:
                wstage[i, pl.ds(0, LANES)] = jnp.where(iota == 0, wb, 0.0)

    for f in range(2):
        h_hbm = h0_hbm if f == 0 else h1_hbm

        @pl.loop(0, CH)
        def _(i):
            for c in range(NBH // LANES):
                stage[i, pl.ds(c * LANES, LANES)] = jnp.zeros(
                    (LANES,), jnp.float32)
            if f == 0:
                wstage[i, pl.ds(0, LANES)] = jnp.zeros((LANES,), jnp.float32)

        @pl.loop(0, RPS // CH)
        def _(r):
            off = sid * RPS + r * CH
            pltpu.sync_copy(stage, feat_acc.at[pl.ds(off, CH)])
            if f == 0:
                pltpu.sync_copy(wstage, w_acc.at[pl.ds(off, CH)])

        plsc.subcore_barrier()

        pltpu.make_async_copy(h_hbm.at[src_v.at[0]], rb0, gsem0).start()

        @pl.loop(0, nchunk // 2)
        def _(t):
            c0 = 2 * t
            c1 = c0 + 1
            pltpu.make_async_copy(h_hbm.at[src_v.at[c1]], rb1, gsem1).start()
            pltpu.make_async_copy(h_hbm.at[src_v.at[c0]], rb0, gsem0).wait()
            _scale(rb0, c0, f)
            pltpu.sync_copy(stage, feat_acc.at[dst_v.at[c0]], add=True)
            if f == 0:
                pltpu.sync_copy(wstage, w_acc.at[dst_v.at[c0]], add=True)
            pltpu.make_async_copy(h_hbm.at[src_v.at[c0 + 2]], rb0,
                                  gsem0).start()
            pltpu.make_async_copy(h_hbm.at[src_v.at[c1]], rb1, gsem1).wait()
            _scale(rb1, c1, f)
            pltpu.sync_copy(stage, feat_acc.at[dst_v.at[c1]], add=True)
            if f == 0:
                pltpu.sync_copy(wstage, w_acc.at[dst_v.at[c1]], add=True)

        pltpu.make_async_copy(h_hbm.at[src_v.at[nchunk]], rb0, gsem0).wait()

        plsc.subcore_barrier()

        @pl.loop(0, RPS // CH)
        def _(r):
            off = sid * RPS + r * CH
            pltpu.sync_copy(feat_acc.at[pl.ds(off, CH)],
                            part_hbm.at[cid, f, pl.ds(off, CH)])
            if f == 0:
                pltpu.sync_copy(w_acc.at[pl.ds(off, CH)],
                                wpart_hbm.at[cid, pl.ds(off, CH)])

        plsc.subcore_barrier()


def _combine_body(part_ref, wpart_ref, h0_ref, h1_ref, as_ref, ad_ref, m_ref,
                  b_ref, p_ref, out_ref):
    P = part_ref[...]
    W = wpart_ref[...]
    num0 = P[0, 0] + P[1, 0]
    num1 = P[0, 1] + P[1, 1]
    wsum = W[0, :, 0] + W[1, :, 0]
    M = m_ref[0, 0] + m_ref[0, 1]
    u = as_ref[...][:, 0] + ad_ref[...][:, 0]
    e = jnp.where(u >= 0.0, u, 0.2 * u)
    wself = jnp.exp(e - M)
    denom = wsum + wself
    o0 = (num0 + wself[:, None] * h0_ref[...]) / denom[:, None]
    o1 = (num1 + wself[:, None] * h1_ref[...]) / denom[:, None]
    o = jnp.concatenate([o0, o1], axis=1) + b_ref[...]
    a = p_ref[0, 0]
    out_ref[...] = jnp.where(o >= 0.0, o, a * o)


def kernel(seq, edge_index, W_fc, b_fc, W_gat, att_src, att_dst, bias_gat,
           prelu_a):
    f32 = jnp.float32

    h0, h1, as2, ad2, mx = pl.pallas_call(
        _mm_body,
        grid=(GRID,),
        in_specs=[
            pl.BlockSpec((RB, FT_IN), lambda i: (i, 0)),
            pl.BlockSpec((NB, FT_IN), lambda i: (0, 0)),
            pl.BlockSpec((1, NB), lambda i: (0, 0)),
            pl.BlockSpec((NB, NB), lambda i: (0, 0)),
            pl.BlockSpec((1, NB), lambda i: (0, 0)),
            pl.BlockSpec((1, NB), lambda i: (0, 0)),
        ],
        out_specs=[
            pl.BlockSpec((RB, NBH), lambda i: (i, 0)),
            pl.BlockSpec((RB, NBH), lambda i: (i, 0)),
            pl.BlockSpec((RB, 1), lambda i: (i, 0)),
            pl.BlockSpec((RB, 1), lambda i: (i, 0)),
            pl.BlockSpec((1, 2), lambda i: (0, 0)),
        ],
        out_shape=[
            jax.ShapeDtypeStruct((N, NBH), f32),
            jax.ShapeDtypeStruct((N, NBH), f32),
            jax.ShapeDtypeStruct((N, 1), f32),
            jax.ShapeDtypeStruct((N, 1), f32),
            jax.ShapeDtypeStruct((1, 2), f32),
        ],
    )(seq, W_fc, b_fc.reshape(1, NB), W_gat, att_src.reshape(1, NB),
      att_dst.reshape(1, NB))

    src = jnp.concatenate(
        [edge_index[0], jnp.zeros((EPPAD - E,), jnp.int32)]).reshape(
            EPPAD // CH, CH)
    dst = jnp.concatenate(
        [edge_index[1], jnp.full((EPPAD - E,), N, jnp.int32)]).reshape(
            EPPAD // CH, CH)
    asrc_p = jnp.pad(as2[:, 0], (0, ROWS - N))
    adst_p = jnp.pad(ad2[:, 0], (0, ROWS - N))
    m16 = jnp.full((LANES,), mx[0, 0] + mx[0, 1], f32)

    def _permute(h):
        return h.reshape(N, 2, NBH // 2).transpose(0, 2, 1).reshape(
            N, NBH).astype(jnp.bfloat16)

    h0p = _permute(h0)
    h1p = _permute(h1)

    mesh = plsc.VectorSubcoreMesh(core_axis_name="c", subcore_axis_name="s")
    cp = pltpu.CompilerParams(needs_layout_passes=False,
                              use_tc_tiling_on_sc=False)
    sc_kernel = pl.kernel(
        _sc_edge_kernel,
        out_type=[
            jax.ShapeDtypeStruct((NC, 2, ROWS, NBH), f32),
            jax.ShapeDtypeStruct((NC, ROWS, LANES), f32),
        ],
        mesh=mesh,
        compiler_params=cp,
        scratch_types=[
            pltpu.VMEM((NCH0 + 1, CH), jnp.int32),
            pltpu.VMEM((NCH0, CH), jnp.int32),
            pltpu.VMEM((LANES,), f32),
            pltpu.VMEM((NCH0 * CH,), f32),
            pltpu.VMEM_SHARED((ROWS, NBH), f32),
            pltpu.VMEM_SHARED((ROWS, LANES), f32),
            pltpu.SemaphoreType.DMA,
            pltpu.SemaphoreType.DMA,
        ],
    )
    part, wpart = sc_kernel(src, dst, asrc_p, adst_p, m16, h0p, h1p)

    out = pl.pallas_call(
        _combine_body,
        grid=(GRID,),
        in_specs=[
            pl.BlockSpec((NC, 2, RB, NBH), lambda i: (0, 0, i, 0)),
            pl.BlockSpec((NC, RB, LANES), lambda i: (0, i, 0)),
            pl.BlockSpec((RB, NBH), lambda i: (i, 0)),
            pl.BlockSpec((RB, NBH), lambda i: (i, 0)),
            pl.BlockSpec((RB, 1), lambda i: (i, 0)),
            pl.BlockSpec((RB, 1), lambda i: (i, 0)),
            pl.BlockSpec((1, 2), lambda i: (0, 0)),
            pl.BlockSpec((1, NB), lambda i: (0, 0)),
            pl.BlockSpec((1, 1), lambda i: (0, 0)),
        ],
        out_specs=pl.BlockSpec((RB, NB), lambda i: (i, 0)),
        out_shape=jax.ShapeDtypeStruct((N, NB), f32),
    )(part, wpart, h0, h1, as2, ad2, mx, bias_gat.reshape(1, NB),
      prelu_a.reshape(1, 1))
    return out


# speedup vs baseline: 14.3313x; 1.0610x over previous
import jax
import jax.numpy as jnp
from jax import lax
from jax.experimental import pallas as pl
from jax.experimental.pallas import tpu as pltpu
from jax.experimental.pallas import tpu_sc as plsc

N = 10000
E = 160000
FT_IN = 512
NB = 256
NBH = NB // 2
NC, NS, LANES = 2, 16, 16
NW = NC * NS
CH = 64
NCH0 = 100
NCH1 = 60
NCHROWS = NS * (NCH0 + NCH1)
EP = NCHROWS * CH
NCHPAD = NCHROWS + NCH0 + 1 - NCH1
EPPAD = 2640 * CH
ROWS = 10240
RPS = ROWS // NS
RB = 1000
GRID = N // RB
RB1 = 2000
GRID1 = N // RB1


def _pack_words(hh):
    lo = lax.bitcast_convert_type(hh[:, :NBH // 2].astype(jnp.bfloat16),
                                  jnp.uint16).astype(jnp.uint32)
    hi = lax.bitcast_convert_type(hh[:, NBH // 2:].astype(jnp.bfloat16),
                                  jnp.uint16).astype(jnp.uint32)
    return (lo | (hi << 16)).astype(jnp.int32)


def _mm_body(seq_ref, wfc_ref, bfc_ref, wgat_ref, asv_ref, adv_ref,
             h0_ref, h1_ref, h0w_ref, h1w_ref, as_ref, ad_ref, mx_ref):
    x = seq_ref[...]
    ret = lax.dot_general(x, wfc_ref[...], (((1,), (1,)), ((), ())),
                          precision=lax.Precision.HIGHEST) + bfc_ref[...]
    h = lax.dot_general(ret, wgat_ref[...], (((1,), (1,)), ((), ())),
                        precision=lax.Precision.HIGHEST)
    a_s = jnp.sum(h * asv_ref[...], axis=1)
    a_d = jnp.sum(h * adv_ref[...], axis=1)
    h0_ref[...] = h[:, :NBH]
    h1_ref[...] = h[:, NBH:]
    h0w_ref[...] = _pack_words(h[:, :NBH])
    h1w_ref[...] = _pack_words(h[:, NBH:])
    as_ref[...] = a_s[:, None]
    ad_ref[...] = a_d[:, None]
    bm = jnp.stack([jnp.max(a_s), jnp.max(a_d)])[None, :]

    @pl.when(pl.program_id(0) == 0)
    def _():
        mx_ref[...] = bm

    @pl.when(pl.program_id(0) > 0)
    def _():
        mx_ref[...] = jnp.maximum(mx_ref[...], bm)


def _sc_edge_kernel(src_hbm, dst_hbm, asrc_hbm, adst_hbm, m_hbm, h0_hbm,
                    h1_hbm, part_hbm, wpart_hbm,
                    src_v, dst_v, m_v, w_v,
                    feat_acc, w_acc, gsem0, gsem1):
    cid = lax.axis_index("c")
    sid = lax.axis_index("s")
    nchunk = jnp.where(cid == 0, NCH0, NCH1)
    row0 = jnp.where(cid == 0, sid * NCH0, NS * NCH0 + sid * NCH1)

    pltpu.sync_copy(src_hbm.at[pl.ds(row0, NCH0 + 1)], src_v)
    pltpu.sync_copy(dst_hbm.at[pl.ds(row0, NCH0)], dst_v)
    pltpu.sync_copy(m_hbm, m_v)

    m16 = m_v[...]
    iota = lax.iota(jnp.int32, LANES)

    def _weights(atab):
        pltpu.sync_copy(asrc_hbm, atab)

        @pl.loop(0, nchunk)
        def _(j):
            @pl.loop(0, CH // LANES)
            def _(k):
                s16 = src_v[j, pl.ds(k * LANES, LANES)]
                w_v[pl.ds(j * CH + k * LANES, LANES)] = plsc.load_gather(
                    atab, [s16])

        pltpu.sync_copy(adst_hbm, atab)

        @pl.loop(0, nchunk)
        def _(j):
            @pl.loop(0, CH // LANES)
            def _(k):
                d16 = dst_v[j, pl.ds(k * LANES, LANES)]
                u = w_v[pl.ds(j * CH + k * LANES, LANES)] + plsc.load_gather(
                    atab, [d16])
                e = jnp.where(u >= 0.0, u, 0.2 * u)
                w_v[pl.ds(j * CH + k * LANES, LANES)] = jnp.exp(e - m16)

    pl.run_scoped(_weights, pltpu.VMEM((ROWS,), jnp.float32))

    def _passes(rb0, rb1, stage, wstage):
        _feature_passes(src_v, dst_v, w_v, h0_hbm, h1_hbm, part_hbm,
                        wpart_hbm, feat_acc, w_acc, gsem0, gsem1, cid, sid,
                        iota, nchunk, rb0, rb1, stage, wstage)

    pl.run_scoped(_passes,
                  pltpu.VMEM((CH, NBH // 2), jnp.int32),
                  pltpu.VMEM((CH, NBH // 2), jnp.int32),
                  pltpu.VMEM((CH, NBH), jnp.float32),
                  pltpu.VMEM((CH, LANES), jnp.float32))


def _feature_passes(src_v, dst_v, w_v, h0_hbm, h1_hbm, part_hbm, wpart_hbm,
                    feat_acc, w_acc, gsem0, gsem1, cid, sid, iota, nchunk,
                    rb0, rb1, stage, wstage):
    def _scale(rb, j, f):
        @pl.loop(0, CH)
        def _(i):
            idx16 = iota * 0 + (j * CH + i)
            wb = plsc.load_gather(w_v, [idx16])
            for c in range(NBH // (2 * LANES)):
                vi = rb[i, pl.ds(c * LANES, LANES)]
                lo = plsc.bitcast(vi << 16, jnp.float32)
                hi = plsc.bitcast(vi & jnp.int32(-65536), jnp.float32)
                stage[i, pl.ds(c * LANES, LANES)] = lo * wb
                stage[i, pl.ds(NBH // 2 + c * LANES, LANES)] = hi * wb
            if f == 0:
                wstage[i, pl.ds(0, LANES)] = jnp.where(iota == 0, wb, 0.0)

    for f in range(2):
        h_hbm = h0_hbm if f == 0 else h1_hbm

        @pl.loop(0, CH)
        def _(i):
            for c in range(NBH // LANES):
                stage[i, pl.ds(c * LANES, LANES)] = jnp.zeros(
                    (LANES,), jnp.float32)
            if f == 0:
                wstage[i, pl.ds(0, LANES)] = jnp.zeros((LANES,), jnp.float32)

        @pl.loop(0, RPS // CH)
        def _(r):
            off = sid * RPS + r * CH
            pltpu.sync_copy(stage, feat_acc.at[pl.ds(off, CH)])
            if f == 0:
                pltpu.sync_copy(wstage, w_acc.at[pl.ds(off, CH)])

        plsc.subcore_barrier()

        pltpu.make_async_copy(h_hbm.at[src_v.at[0]], rb0, gsem0).start()

        @pl.loop(0, nchunk // 2)
        def _(t):
            c0 = 2 * t
            c1 = c0 + 1
            pltpu.make_async_copy(h_hbm.at[src_v.at[c1]], rb1, gsem1).start()
            pltpu.make_async_copy(h_hbm.at[src_v.at[c0]], rb0, gsem0).wait()
            _scale(rb0, c0, f)
            pltpu.sync_copy(stage, feat_acc.at[dst_v.at[c0]], add=True)
            if f == 0:
                pltpu.sync_copy(wstage, w_acc.at[dst_v.at[c0]], add=True)
            pltpu.make_async_copy(h_hbm.at[src_v.at[c0 + 2]], rb0,
                                  gsem0).start()
            pltpu.make_async_copy(h_hbm.at[src_v.at[c1]], rb1, gsem1).wait()
            _scale(rb1, c1, f)
            pltpu.sync_copy(stage, feat_acc.at[dst_v.at[c1]], add=True)
            if f == 0:
                pltpu.sync_copy(wstage, w_acc.at[dst_v.at[c1]], add=True)

        pltpu.make_async_copy(h_hbm.at[src_v.at[nchunk]], rb0, gsem0).wait()

        plsc.subcore_barrier()

        @pl.loop(0, RPS // CH)
        def _(r):
            off = sid * RPS + r * CH
            pltpu.sync_copy(feat_acc.at[pl.ds(off, CH)],
                            part_hbm.at[cid, f, pl.ds(off, CH)])
            if f == 0:
                pltpu.sync_copy(w_acc.at[pl.ds(off, CH)],
                                wpart_hbm.at[cid, pl.ds(off, CH)])

        plsc.subcore_barrier()


def _combine_body(part_ref, wpart_ref, h0_ref, h1_ref, as_ref, ad_ref, m_ref,
                  b_ref, p_ref, out_ref):
    P = part_ref[...]
    W = wpart_ref[...]
    num0 = P[0, 0] + P[1, 0]
    num1 = P[0, 1] + P[1, 1]
    wsum = W[0, :, 0] + W[1, :, 0]
    M = m_ref[0, 0] + m_ref[0, 1]
    u = as_ref[...][:, 0] + ad_ref[...][:, 0]
    e = jnp.where(u >= 0.0, u, 0.2 * u)
    wself = jnp.exp(e - M)
    denom = wsum + wself
    o0 = (num0 + wself[:, None] * h0_ref[...]) / denom[:, None]
    o1 = (num1 + wself[:, None] * h1_ref[...]) / denom[:, None]
    o = jnp.concatenate([o0, o1], axis=1) + b_ref[...]
    a = p_ref[0, 0]
    out_ref[...] = jnp.where(o >= 0.0, o, a * o)


def kernel(seq, edge_index, W_fc, b_fc, W_gat, att_src, att_dst, bias_gat,
           prelu_a):
    f32 = jnp.float32

    h0, h1, h0w, h1w, as2, ad2, mx = pl.pallas_call(
        _mm_body,
        grid=(GRID1,),
        in_specs=[
            pl.BlockSpec((RB1, FT_IN), lambda i: (i, 0)),
            pl.BlockSpec((NB, FT_IN), lambda i: (0, 0)),
            pl.BlockSpec((1, NB), lambda i: (0, 0)),
            pl.BlockSpec((NB, NB), lambda i: (0, 0)),
            pl.BlockSpec((1, NB), lambda i: (0, 0)),
            pl.BlockSpec((1, NB), lambda i: (0, 0)),
        ],
        out_specs=[
            pl.BlockSpec((RB1, NBH), lambda i: (i, 0)),
            pl.BlockSpec((RB1, NBH), lambda i: (i, 0)),
            pl.BlockSpec((RB1, NBH // 2), lambda i: (i, 0)),
            pl.BlockSpec((RB1, NBH // 2), lambda i: (i, 0)),
            pl.BlockSpec((RB1, 1), lambda i: (i, 0)),
            pl.BlockSpec((RB1, 1), lambda i: (i, 0)),
            pl.BlockSpec((1, 2), lambda i: (0, 0)),
        ],
        out_shape=[
            jax.ShapeDtypeStruct((N, NBH), f32),
            jax.ShapeDtypeStruct((N, NBH), f32),
            jax.ShapeDtypeStruct((N, NBH // 2), jnp.int32),
            jax.ShapeDtypeStruct((N, NBH // 2), jnp.int32),
            jax.ShapeDtypeStruct((ROWS, 1), f32),
            jax.ShapeDtypeStruct((ROWS, 1), f32),
            jax.ShapeDtypeStruct((1, 2), f32),
        ],
    )(seq, W_fc, b_fc.reshape(1, NB), W_gat, att_src.reshape(1, NB),
      att_dst.reshape(1, NB))

    src = jnp.concatenate(
        [edge_index[0], jnp.zeros((EPPAD - E,), jnp.int32)]).reshape(
            EPPAD // CH, CH)
    dst = jnp.concatenate(
        [edge_index[1], jnp.full((EPPAD - E,), N, jnp.int32)]).reshape(
            EPPAD // CH, CH)
    asrc_p = as2.reshape(ROWS)
    adst_p = ad2.reshape(ROWS)
    m16 = jnp.full((LANES,), mx[0, 0] + mx[0, 1], f32)

    mesh = plsc.VectorSubcoreMesh(core_axis_name="c", subcore_axis_name="s")
    cp = pltpu.CompilerParams(needs_layout_passes=False,
                              use_tc_tiling_on_sc=False)
    sc_kernel = pl.kernel(
        _sc_edge_kernel,
        out_type=[
            jax.ShapeDtypeStruct((NC, 2, ROWS, NBH), f32),
            jax.ShapeDtypeStruct((NC, ROWS, LANES), f32),
        ],
        mesh=mesh,
        compiler_params=cp,
        scratch_types=[
            pltpu.VMEM((NCH0 + 1, CH), jnp.int32),
            pltpu.VMEM((NCH0, CH), jnp.int32),
            pltpu.VMEM((LANES,), f32),
            pltpu.VMEM((NCH0 * CH,), f32),
            pltpu.VMEM_SHARED((ROWS, NBH), f32),
            pltpu.VMEM_SHARED((ROWS, LANES), f32),
            pltpu.SemaphoreType.DMA,
            pltpu.SemaphoreType.DMA,
        ],
    )
    part, wpart = sc_kernel(src, dst, asrc_p, adst_p, m16, h0w, h1w)

    out = pl.pallas_call(
        _combine_body,
        grid=(GRID,),
        in_specs=[
            pl.BlockSpec((NC, 2, RB, NBH), lambda i: (0, 0, i, 0)),
            pl.BlockSpec((NC, RB, LANES), lambda i: (0, i, 0)),
            pl.BlockSpec((RB, NBH), lambda i: (i, 0)),
            pl.BlockSpec((RB, NBH), lambda i: (i, 0)),
            pl.BlockSpec((RB, 1), lambda i: (i, 0)),
            pl.BlockSpec((RB, 1), lambda i: (i, 0)),
            pl.BlockSpec((1, 2), lambda i: (0, 0)),
            pl.BlockSpec((1, NB), lambda i: (0, 0)),
            pl.BlockSpec((1, 1), lambda i: (0, 0)),
        ],
        out_specs=pl.BlockSpec((RB, NB), lambda i: (i, 0)),
        out_shape=jax.ShapeDtypeStruct((N, NB), f32),
    )(part, wpart, h0, h1, as2, ad2, mx, bias_gat.reshape(1, NB),
      prelu_a.reshape(1, 1))
    return out


# speedup vs baseline: 15.4155x; 1.0757x over previous
import jax
import jax.numpy as jnp
from jax import lax
from jax.experimental import pallas as pl
from jax.experimental.pallas import tpu as pltpu
from jax.experimental.pallas import tpu_sc as plsc

N = 10000
E = 160000
FT_IN = 512
NB = 256
NBH = NB // 2
NC, NS, LANES = 2, 16, 16
NW = NC * NS
CH = 64
NCH0 = 96
NCH1 = 64
NCHROWS = NS * (NCH0 + NCH1)
EP = NCHROWS * CH
NCHPAD = NCHROWS + NCH0 + 1 - NCH1
EPPAD = 2640 * CH
ROWS = 10240
RPS = ROWS // NS
RB = 1000
GRID = N // RB
RB1 = 2000
GRID1 = N // RB1


def _pack_words(hh):
    lo = lax.bitcast_convert_type(hh[:, :NBH // 2].astype(jnp.bfloat16),
                                  jnp.uint16).astype(jnp.uint32)
    hi = lax.bitcast_convert_type(hh[:, NBH // 2:].astype(jnp.bfloat16),
                                  jnp.uint16).astype(jnp.uint32)
    return (lo | (hi << 16)).astype(jnp.int32)


def _dot3(x, w):
    f32 = jnp.float32
    xh = x.astype(jnp.bfloat16)
    xl = (x - xh.astype(f32)).astype(jnp.bfloat16)
    wh = w.astype(jnp.bfloat16)
    wl = (w - wh.astype(f32)).astype(jnp.bfloat16)

    def d(a, b):
        return lax.dot_general(a, b, (((1,), (1,)), ((), ())),
                               preferred_element_type=f32)

    return d(xh, wh) + d(xl, wh) + d(xh, wl)


def _mm_body(seq_ref, wfc_ref, bfc_ref, wgat_ref, asv_ref, adv_ref,
             h0_ref, h1_ref, h0w_ref, h1w_ref, as_ref, ad_ref, mx_ref):
    x = seq_ref[...]
    ret = _dot3(x, wfc_ref[...]) + bfc_ref[...]
    h = _dot3(ret, wgat_ref[...])
    a_s = jnp.sum(h * asv_ref[...], axis=1)
    a_d = jnp.sum(h * adv_ref[...], axis=1)
    h0_ref[...] = h[:, :NBH]
    h1_ref[...] = h[:, NBH:]
    h0w_ref[...] = _pack_words(h[:, :NBH])
    h1w_ref[...] = _pack_words(h[:, NBH:])
    as_ref[...] = a_s[:, None]
    ad_ref[...] = a_d[:, None]
    bm = jnp.stack([jnp.max(a_s), jnp.max(a_d)])[None, :]

    @pl.when(pl.program_id(0) == 0)
    def _():
        mx_ref[...] = bm

    @pl.when(pl.program_id(0) > 0)
    def _():
        mx_ref[...] = jnp.maximum(mx_ref[...], bm)


def _sc_edge_kernel(src_hbm, dst_hbm, asrc_hbm, adst_hbm, m_hbm, h0_hbm,
                    h1_hbm, part_hbm, wpart_hbm,
                    src_v, dst_v, m_v, w_v,
                    feat_acc, w_acc, gsem0, gsem1):
    cid = lax.axis_index("c")
    sid = lax.axis_index("s")
    nchunk = jnp.where(cid == 0, NCH0, NCH1)
    row0 = jnp.where(cid == 0, sid * NCH0, NS * NCH0 + sid * NCH1)

    pltpu.sync_copy(src_hbm.at[pl.ds(row0, NCH0 + 1)], src_v)
    pltpu.sync_copy(dst_hbm.at[pl.ds(row0, NCH0)], dst_v)
    pltpu.sync_copy(m_hbm, m_v)

    m16 = m_v[...]
    iota = lax.iota(jnp.int32, LANES)

    def _weights(atab):
        pltpu.sync_copy(asrc_hbm, atab)

        @pl.loop(0, nchunk)
        def _(j):
            @pl.loop(0, CH // LANES)
            def _(k):
                s16 = src_v[j, pl.ds(k * LANES, LANES)]
                w_v[pl.ds(j * CH + k * LANES, LANES)] = plsc.load_gather(
                    atab, [s16])

        pltpu.sync_copy(adst_hbm, atab)

        @pl.loop(0, nchunk)
        def _(j):
            @pl.loop(0, CH // LANES)
            def _(k):
                d16 = dst_v[j, pl.ds(k * LANES, LANES)]
                u = w_v[pl.ds(j * CH + k * LANES, LANES)] + plsc.load_gather(
                    atab, [d16])
                e = jnp.where(u >= 0.0, u, 0.2 * u)
                w_v[pl.ds(j * CH + k * LANES, LANES)] = jnp.exp(e - m16)

    pl.run_scoped(_weights, pltpu.VMEM((ROWS,), jnp.float32))

    def _passes(rb0, rb1, stage, wstage):
        _feature_passes(src_v, dst_v, w_v, h0_hbm, h1_hbm, part_hbm,
                        wpart_hbm, feat_acc, w_acc, gsem0, gsem1, cid, sid,
                        iota, nchunk, rb0, rb1, stage, wstage)

    pl.run_scoped(_passes,
                  pltpu.VMEM((CH, NBH // 2), jnp.int32),
                  pltpu.VMEM((CH, NBH // 2), jnp.int32),
                  pltpu.VMEM((CH, NBH), jnp.float32),
                  pltpu.VMEM((CH, LANES), jnp.float32))


def _feature_passes(src_v, dst_v, w_v, h0_hbm, h1_hbm, part_hbm, wpart_hbm,
                    feat_acc, w_acc, gsem0, gsem1, cid, sid, iota, nchunk,
                    rb0, rb1, stage, wstage):
    def _scale(rb, j, f):
        @pl.loop(0, CH)
        def _(i):
            idx16 = iota * 0 + (j * CH + i)
            wb = plsc.load_gather(w_v, [idx16])
            for c in range(NBH // (2 * LANES)):
                vi = rb[i, pl.ds(c * LANES, LANES)]
                lo = plsc.bitcast(vi << 16, jnp.float32)
                hi = plsc.bitcast(vi & jnp.int32(-65536), jnp.float32)
                stage[i, pl.ds(c * LANES, LANES)] = lo * wb
                stage[i, pl.ds(NBH // 2 + c * LANES, LANES)] = hi * wb
            if f == 0:
                wstage[i, pl.ds(0, LANES)] = jnp.where(iota == 0, wb, 0.0)

    for f in range(2):
        h_hbm = h0_hbm if f == 0 else h1_hbm

        @pl.loop(0, CH)
        def _(i):
            for c in range(NBH // LANES):
                stage[i, pl.ds(c * LANES, LANES)] = jnp.zeros(
                    (LANES,), jnp.float32)
            if f == 0:
                wstage[i, pl.ds(0, LANES)] = jnp.zeros((LANES,), jnp.float32)

        @pl.loop(0, RPS // CH)
        def _(r):
            off = sid * RPS + r * CH
            pltpu.sync_copy(stage, feat_acc.at[pl.ds(off, CH)])
            if f == 0:
                pltpu.sync_copy(wstage, w_acc.at[pl.ds(off, CH)])

        plsc.subcore_barrier()

        pltpu.make_async_copy(h_hbm.at[src_v.at[0]], rb0, gsem0).start()

        @pl.loop(0, nchunk // 2)
        def _(t):
            c0 = 2 * t
            c1 = c0 + 1
            pltpu.make_async_copy(h_hbm.at[src_v.at[c1]], rb1, gsem1).start()
            pltpu.make_async_copy(h_hbm.at[src_v.at[c0]], rb0, gsem0).wait()
            _scale(rb0, c0, f)
            pltpu.sync_copy(stage, feat_acc.at[dst_v.at[c0]], add=True)
            if f == 0:
                pltpu.sync_copy(wstage, w_acc.at[dst_v.at[c0]], add=True)
            pltpu.make_async_copy(h_hbm.at[src_v.at[c0 + 2]], rb0,
                                  gsem0).start()
            pltpu.make_async_copy(h_hbm.at[src_v.at[c1]], rb1, gsem1).wait()
            _scale(rb1, c1, f)
            pltpu.sync_copy(stage, feat_acc.at[dst_v.at[c1]], add=True)
            if f == 0:
                pltpu.sync_copy(wstage, w_acc.at[dst_v.at[c1]], add=True)

        pltpu.make_async_copy(h_hbm.at[src_v.at[nchunk]], rb0, gsem0).wait()

        plsc.subcore_barrier()

        @pl.loop(0, RPS // CH)
        def _(r):
            off = sid * RPS + r * CH
            pltpu.sync_copy(feat_acc.at[pl.ds(off, CH)],
                            part_hbm.at[cid, f, pl.ds(off, CH)])
            if f == 0:
                pltpu.sync_copy(w_acc.at[pl.ds(off, CH)],
                                wpart_hbm.at[cid, pl.ds(off, CH)])

        plsc.subcore_barrier()


def _combine_body(part_ref, wpart_ref, h0_ref, h1_ref, as_ref, ad_ref, m_ref,
                  b_ref, p_ref, out_ref):
    P = part_ref[...]
    W = wpart_ref[...]
    num0 = P[0, 0] + P[1, 0]
    num1 = P[0, 1] + P[1, 1]
    wsum = W[0, :, 0] + W[1, :, 0]
    M = m_ref[0, 0] + m_ref[0, 1]
    u = as_ref[...][:, 0] + ad_ref[...][:, 0]
    e = jnp.where(u >= 0.0, u, 0.2 * u)
    wself = jnp.exp(e - M)
    denom = wsum + wself
    o0 = (num0 + wself[:, None] * h0_ref[...]) / denom[:, None]
    o1 = (num1 + wself[:, None] * h1_ref[...]) / denom[:, None]
    o = jnp.concatenate([o0, o1], axis=1) + b_ref[...]
    a = p_ref[0, 0]
    out_ref[...] = jnp.where(o >= 0.0, o, a * o)


def kernel(seq, edge_index, W_fc, b_fc, W_gat, att_src, att_dst, bias_gat,
           prelu_a):
    f32 = jnp.float32

    h0, h1, h0w, h1w, as2, ad2, mx = pl.pallas_call(
        _mm_body,
        grid=(GRID1,),
        in_specs=[
            pl.BlockSpec((RB1, FT_IN), lambda i: (i, 0)),
            pl.BlockSpec((NB, FT_IN), lambda i: (0, 0)),
            pl.BlockSpec((1, NB), lambda i: (0, 0)),
            pl.BlockSpec((NB, NB), lambda i: (0, 0)),
            pl.BlockSpec((1, NB), lambda i: (0, 0)),
            pl.BlockSpec((1, NB), lambda i: (0, 0)),
        ],
        out_specs=[
            pl.BlockSpec((RB1, NBH), lambda i: (i, 0)),
            pl.BlockSpec((RB1, NBH), lambda i: (i, 0)),
            pl.BlockSpec((RB1, NBH // 2), lambda i: (i, 0)),
            pl.BlockSpec((RB1, NBH // 2), lambda i: (i, 0)),
            pl.BlockSpec((RB1, 1), lambda i: (i, 0)),
            pl.BlockSpec((RB1, 1), lambda i: (i, 0)),
            pl.BlockSpec((1, 2), lambda i: (0, 0)),
        ],
        out_shape=[
            jax.ShapeDtypeStruct((N, NBH), f32),
            jax.ShapeDtypeStruct((N, NBH), f32),
            jax.ShapeDtypeStruct((N, NBH // 2), jnp.int32),
            jax.ShapeDtypeStruct((N, NBH // 2), jnp.int32),
            jax.ShapeDtypeStruct((ROWS, 1), f32),
            jax.ShapeDtypeStruct((ROWS, 1), f32),
            jax.ShapeDtypeStruct((1, 2), f32),
        ],
    )(seq, W_fc, b_fc.reshape(1, NB), W_gat, att_src.reshape(1, NB),
      att_dst.reshape(1, NB))

    src = jnp.concatenate(
        [edge_index[0], jnp.zeros((EPPAD - E,), jnp.int32)]).reshape(
            EPPAD // CH, CH)
    dst = jnp.concatenate(
        [edge_index[1], jnp.full((EPPAD - E,), N, jnp.int32)]).reshape(
            EPPAD // CH, CH)
    asrc_p = as2.reshape(ROWS)
    adst_p = ad2.reshape(ROWS)
    m16 = jnp.full((LANES,), mx[0, 0] + mx[0, 1], f32)

    mesh = plsc.VectorSubcoreMesh(core_axis_name="c", subcore_axis_name="s")
    cp = pltpu.CompilerParams(needs_layout_passes=False,
                              use_tc_tiling_on_sc=False)
    sc_kernel = pl.kernel(
        _sc_edge_kernel,
        out_type=[
            jax.ShapeDtypeStruct((NC, 2, ROWS, NBH), f32),
            jax.ShapeDtypeStruct((NC, ROWS, LANES), f32),
        ],
        mesh=mesh,
        compiler_params=cp,
        scratch_types=[
            pltpu.VMEM((NCH0 + 1, CH), jnp.int32),
            pltpu.VMEM((NCH0, CH), jnp.int32),
            pltpu.VMEM((LANES,), f32),
            pltpu.VMEM((NCH0 * CH,), f32),
            pltpu.VMEM_SHARED((ROWS, NBH), f32),
            pltpu.VMEM_SHARED((ROWS, LANES), f32),
            pltpu.SemaphoreType.DMA,
            pltpu.SemaphoreType.DMA,
        ],
    )
    part, wpart = sc_kernel(src, dst, asrc_p, adst_p, m16, h0w, h1w)

    out = pl.pallas_call(
        _combine_body,
        grid=(GRID,),
        in_specs=[
            pl.BlockSpec((NC, 2, RB, NBH), lambda i: (0, 0, i, 0)),
            pl.BlockSpec((NC, RB, LANES), lambda i: (0, i, 0)),
            pl.BlockSpec((RB, NBH), lambda i: (i, 0)),
            pl.BlockSpec((RB, NBH), lambda i: (i, 0)),
            pl.BlockSpec((RB, 1), lambda i: (i, 0)),
            pl.BlockSpec((RB, 1), lambda i: (i, 0)),
            pl.BlockSpec((1, 2), lambda i: (0, 0)),
            pl.BlockSpec((1, NB), lambda i: (0, 0)),
            pl.BlockSpec((1, 1), lambda i: (0, 0)),
        ],
        out_specs=pl.BlockSpec((RB, NB), lambda i: (i, 0)),
        out_shape=jax.ShapeDtypeStruct((N, NB), f32),
    )(part, wpart, h0, h1, as2, ad2, mx, bias_gat.reshape(1, NB),
      prelu_a.reshape(1, 1))
    return out


# speedup vs baseline: 15.7113x; 1.0192x over previous
import jax
import jax.numpy as jnp
from jax import lax
from jax.experimental import pallas as pl
from jax.experimental.pallas import tpu as pltpu
from jax.experimental.pallas import tpu_sc as plsc

N = 10000
E = 160000
FT_IN = 512
NB = 256
NBH = NB // 2
NC, NS, LANES = 2, 16, 16
NW = NC * NS
CH = 64
NCH0 = 96
NCH1 = 64
NCHROWS = NS * (NCH0 + NCH1)
EP = NCHROWS * CH
NCHPAD = NCHROWS + NCH0 + 1 - NCH1
EPPAD = 2640 * CH
ROWS = 10240
RPS = ROWS // NS
RB = 1000
GRID = N // RB
RB1 = 2000
GRID1 = N // RB1


def _pack_words(hh):
    lo = lax.bitcast_convert_type(hh[:, :NBH // 2].astype(jnp.bfloat16),
                                  jnp.uint16).astype(jnp.uint32)
    hi = lax.bitcast_convert_type(hh[:, NBH // 2:].astype(jnp.bfloat16),
                                  jnp.uint16).astype(jnp.uint32)
    return (lo | (hi << 16)).astype(jnp.int32)


def _dot3(x, w):
    f32 = jnp.float32
    xh = x.astype(jnp.bfloat16)
    xl = (x - xh.astype(f32)).astype(jnp.bfloat16)
    wh = w.astype(jnp.bfloat16)
    wl = (w - wh.astype(f32)).astype(jnp.bfloat16)

    def d(a, b):
        return lax.dot_general(a, b, (((1,), (1,)), ((), ())),
                               preferred_element_type=f32)

    return d(xh, wh) + d(xl, wh) + d(xh, wl)


def _mm_body(seq_ref, wfc_ref, bfc_ref, wgat_ref, asv_ref, adv_ref,
             h0_ref, h1_ref, h0w_ref, h1w_ref, as_ref, ad_ref, mx_ref):
    x = seq_ref[...]
    ret = _dot3(x, wfc_ref[...]) + bfc_ref[...]
    h = _dot3(ret, wgat_ref[...])
    a_s = jnp.sum(h * asv_ref[...], axis=1)
    a_d = jnp.sum(h * adv_ref[...], axis=1)
    h0_ref[...] = h[:, :NBH]
    h1_ref[...] = h[:, NBH:]
    h0w_ref[...] = _pack_words(h[:, :NBH])
    h1w_ref[...] = _pack_words(h[:, NBH:])
    as_ref[...] = a_s[:, None]
    ad_ref[...] = a_d[:, None]
    bm = jnp.stack([jnp.max(a_s), jnp.max(a_d)])[None, :]

    @pl.when(pl.program_id(0) == 0)
    def _():
        mx_ref[...] = bm

    @pl.when(pl.program_id(0) > 0)
    def _():
        mx_ref[...] = jnp.maximum(mx_ref[...], bm)


def _sc_edge_kernel(src_hbm, dst_hbm, asrc_hbm, adst_hbm, m_hbm, h0_hbm,
                    h1_hbm, part_hbm, wpart_hbm,
                    src_v, dst_v, m_v, w_v,
                    feat_acc, w_acc, gsem0, gsem1):
    cid = lax.axis_index("c")
    sid = lax.axis_index("s")
    nchunk = jnp.where(cid == 0, NCH0, NCH1)
    row0 = jnp.where(cid == 0, sid * NCH0, NS * NCH0 + sid * NCH1)

    pltpu.sync_copy(src_hbm.at[pl.ds(row0, NCH0 + 1)], src_v)
    pltpu.sync_copy(dst_hbm.at[pl.ds(row0, NCH0)], dst_v)
    pltpu.sync_copy(m_hbm, m_v)

    m16 = m_v[...]
    iota = lax.iota(jnp.int32, LANES)

    def _weights(atab):
        pltpu.sync_copy(asrc_hbm, atab)

        @pl.loop(0, nchunk)
        def _(j):
            @pl.loop(0, CH // LANES)
            def _(k):
                s16 = src_v[j, pl.ds(k * LANES, LANES)]
                w_v[pl.ds(j * CH + k * LANES, LANES)] = plsc.load_gather(
                    atab, [s16])

        pltpu.sync_copy(adst_hbm, atab)

        @pl.loop(0, nchunk)
        def _(j):
            @pl.loop(0, CH // LANES)
            def _(k):
                d16 = dst_v[j, pl.ds(k * LANES, LANES)]
                u = w_v[pl.ds(j * CH + k * LANES, LANES)] + plsc.load_gather(
                    atab, [d16])
                e = jnp.where(u >= 0.0, u, 0.2 * u)
                w_v[pl.ds(j * CH + k * LANES, LANES)] = jnp.exp(e - m16)

    pl.run_scoped(_weights, pltpu.VMEM((ROWS,), jnp.float32))

    def _passes(rb0, rb1, stage, wstage):
        _feature_passes(src_v, dst_v, w_v, h0_hbm, h1_hbm, part_hbm,
                        wpart_hbm, feat_acc, w_acc, gsem0, gsem1, cid, sid,
                        iota, nchunk, rb0, rb1, stage, wstage)

    pl.run_scoped(_passes,
                  pltpu.VMEM((CH, NBH // 2), jnp.int32),
                  pltpu.VMEM((CH, NBH // 2), jnp.int32),
                  pltpu.VMEM((CH, NBH), jnp.float32),
                  pltpu.VMEM((CH, LANES), jnp.float32))


def _feature_passes(src_v, dst_v, w_v, h0_hbm, h1_hbm, part_hbm, wpart_hbm,
                    feat_acc, w_acc, gsem0, gsem1, cid, sid, iota, nchunk,
                    rb0, rb1, stage, wstage):
    def _scale(rb, j, f):
        @plsc.parallel_loop(0, CH, unroll=4)
        def _(i):
            idx16 = iota * 0 + (j * CH + i)
            wb = plsc.load_gather(w_v, [idx16])
            for c in range(NBH // (2 * LANES)):
                vi = rb[i, pl.ds(c * LANES, LANES)]
                lo = plsc.bitcast(vi << 16, jnp.float32)
                hi = plsc.bitcast(vi & jnp.int32(-65536), jnp.float32)
                stage[i, pl.ds(c * LANES, LANES)] = lo * wb
                stage[i, pl.ds(NBH // 2 + c * LANES, LANES)] = hi * wb
            if f == 0:
                wstage[i, pl.ds(0, LANES)] = jnp.where(iota == 0, wb, 0.0)

    for f in range(2):
        h_hbm = h0_hbm if f == 0 else h1_hbm

        @pl.loop(0, CH)
        def _(i):
            for c in range(NBH // LANES):
                stage[i, pl.ds(c * LANES, LANES)] = jnp.zeros(
                    (LANES,), jnp.float32)
            if f == 0:
                wstage[i, pl.ds(0, LANES)] = jnp.zeros((LANES,), jnp.float32)

        @pl.loop(0, RPS // CH)
        def _(r):
            off = sid * RPS + r * CH
            pltpu.sync_copy(stage, feat_acc.at[pl.ds(off, CH)])
            if f == 0:
                pltpu.sync_copy(wstage, w_acc.at[pl.ds(off, CH)])

        plsc.subcore_barrier()

        pltpu.make_async_copy(h_hbm.at[src_v.at[0]], rb0, gsem0).start()

        @pl.loop(0, nchunk // 2)
        def _(t):
            c0 = 2 * t
            c1 = c0 + 1
            pltpu.make_async_copy(h_hbm.at[src_v.at[c1]], rb1, gsem1).start()
            pltpu.make_async_copy(h_hbm.at[src_v.at[c0]], rb0, gsem0).wait()
            _scale(rb0, c0, f)
            pltpu.sync_copy(stage, feat_acc.at[dst_v.at[c0]], add=True)
            if f == 0:
                pltpu.sync_copy(wstage, w_acc.at[dst_v.at[c0]], add=True)
            pltpu.make_async_copy(h_hbm.at[src_v.at[c0 + 2]], rb0,
                                  gsem0).start()
            pltpu.make_async_copy(h_hbm.at[src_v.at[c1]], rb1, gsem1).wait()
            _scale(rb1, c1, f)
            pltpu.sync_copy(stage, feat_acc.at[dst_v.at[c1]], add=True)
            if f == 0:
                pltpu.sync_copy(wstage, w_acc.at[dst_v.at[c1]], add=True)

        pltpu.make_async_copy(h_hbm.at[src_v.at[nchunk]], rb0, gsem0).wait()

        plsc.subcore_barrier()

        @pl.loop(0, RPS // CH)
        def _(r):
            off = sid * RPS + r * CH
            pltpu.sync_copy(feat_acc.at[pl.ds(off, CH)],
                            part_hbm.at[cid, f, pl.ds(off, CH)])
            if f == 0:
                pltpu.sync_copy(w_acc.at[pl.ds(off, CH)],
                                wpart_hbm.at[cid, pl.ds(off, CH)])

        plsc.subcore_barrier()


def _combine_body(part_ref, wpart_ref, h0_ref, h1_ref, as_ref, ad_ref, m_ref,
                  b_ref, p_ref, out_ref):
    P = part_ref[...]
    W = wpart_ref[...]
    num0 = P[0, 0] + P[1, 0]
    num1 = P[0, 1] + P[1, 1]
    wsum = W[0, :, 0] + W[1, :, 0]
    M = m_ref[0, 0] + m_ref[0, 1]
    u = as_ref[...][:, 0] + ad_ref[...][:, 0]
    e = jnp.where(u >= 0.0, u, 0.2 * u)
    wself = jnp.exp(e - M)
    denom = wsum + wself
    o0 = (num0 + wself[:, None] * h0_ref[...]) / denom[:, None]
    o1 = (num1 + wself[:, None] * h1_ref[...]) / denom[:, None]
    o = jnp.concatenate([o0, o1], axis=1) + b_ref[...]
    a = p_ref[0, 0]
    out_ref[...] = jnp.where(o >= 0.0, o, a * o)


def kernel(seq, edge_index, W_fc, b_fc, W_gat, att_src, att_dst, bias_gat,
           prelu_a):
    f32 = jnp.float32

    h0, h1, h0w, h1w, as2, ad2, mx = pl.pallas_call(
        _mm_body,
        grid=(GRID1,),
        in_specs=[
            pl.BlockSpec((RB1, FT_IN), lambda i: (i, 0)),
            pl.BlockSpec((NB, FT_IN), lambda i: (0, 0)),
            pl.BlockSpec((1, NB), lambda i: (0, 0)),
            pl.BlockSpec((NB, NB), lambda i: (0, 0)),
            pl.BlockSpec((1, NB), lambda i: (0, 0)),
            pl.BlockSpec((1, NB), lambda i: (0, 0)),
        ],
        out_specs=[
            pl.BlockSpec((RB1, NBH), lambda i: (i, 0)),
            pl.BlockSpec((RB1, NBH), lambda i: (i, 0)),
            pl.BlockSpec((RB1, NBH // 2), lambda i: (i, 0)),
            pl.BlockSpec((RB1, NBH // 2), lambda i: (i, 0)),
            pl.BlockSpec((RB1, 1), lambda i: (i, 0)),
            pl.BlockSpec((RB1, 1), lambda i: (i, 0)),
            pl.BlockSpec((1, 2), lambda i: (0, 0)),
        ],
        out_shape=[
            jax.ShapeDtypeStruct((N, NBH), f32),
            jax.ShapeDtypeStruct((N, NBH), f32),
            jax.ShapeDtypeStruct((N, NBH // 2), jnp.int32),
            jax.ShapeDtypeStruct((N, NBH // 2), jnp.int32),
            jax.ShapeDtypeStruct((ROWS, 1), f32),
            jax.ShapeDtypeStruct((ROWS, 1), f32),
            jax.ShapeDtypeStruct((1, 2), f32),
        ],
    )(seq, W_fc, b_fc.reshape(1, NB), W_gat, att_src.reshape(1, NB),
      att_dst.reshape(1, NB))

    src = jnp.concatenate(
        [edge_index[0], jnp.zeros((EPPAD - E,), jnp.int32)]).reshape(
            EPPAD // CH, CH)
    dst = jnp.concatenate(
        [edge_index[1], jnp.full((EPPAD - E,), N, jnp.int32)]).reshape(
            EPPAD // CH, CH)
    asrc_p = as2.reshape(ROWS)
    adst_p = ad2.reshape(ROWS)
    m16 = jnp.full((LANES,), mx[0, 0] + mx[0, 1], f32)

    mesh = plsc.VectorSubcoreMesh(core_axis_name="c", subcore_axis_name="s")
    cp = pltpu.CompilerParams(needs_layout_passes=False,
                              use_tc_tiling_on_sc=False)
    sc_kernel = pl.kernel(
        _sc_edge_kernel,
        out_type=[
            jax.ShapeDtypeStruct((NC, 2, ROWS, NBH), f32),
            jax.ShapeDtypeStruct((NC, ROWS, LANES), f32),
        ],
        mesh=mesh,
        compiler_params=cp,
        scratch_types=[
            pltpu.VMEM((NCH0 + 1, CH), jnp.int32),
            pltpu.VMEM((NCH0, CH), jnp.int32),
            pltpu.VMEM((LANES,), f32),
            pltpu.VMEM((NCH0 * CH,), f32),
            pltpu.VMEM_SHARED((ROWS, NBH), f32),
            pltpu.VMEM_SHARED((ROWS, LANES), f32),
            pltpu.SemaphoreType.DMA,
            pltpu.SemaphoreType.DMA,
        ],
    )
    part, wpart = sc_kernel(src, dst, asrc_p, adst_p, m16, h0w, h1w)

    out = pl.pallas_call(
        _combine_body,
        grid=(GRID,),
        in_specs=[
            pl.BlockSpec((NC, 2, RB, NBH), lambda i: (0, 0, i, 0)),
            pl.BlockSpec((NC, RB, LANES), lambda i: (0, i, 0)),
            pl.BlockSpec((RB, NBH), lambda i: (i, 0)),
            pl.BlockSpec((RB, NBH), lambda i: (i, 0)),
            pl.BlockSpec((RB, 1), lambda i: (i, 0)),
            pl.BlockSpec((RB, 1), lambda i: (i, 0)),
            pl.BlockSpec((1, 2), lambda i: (0, 0)),
            pl.BlockSpec((1, NB), lambda i: (0, 0)),
            pl.BlockSpec((1, 1), lambda i: (0, 0)),
        ],
        out_specs=pl.BlockSpec((RB, NB), lambda i: (i, 0)),
        out_shape=jax.ShapeDtypeStruct((N, NB), f32),
    )(part, wpart, h0, h1, as2, ad2, mx, bias_gat.reshape(1, NB),
      prelu_a.reshape(1, 1))
    return out


# speedup vs baseline: 15.7759x; 1.0041x over previous
import jax
import jax.numpy as jnp
from jax import lax
from jax.experimental import pallas as pl
from jax.experimental.pallas import tpu as pltpu
from jax.experimental.pallas import tpu_sc as plsc

N = 10000
E = 160000
FT_IN = 512
NB = 256
NBH = NB // 2
NC, NS, LANES = 2, 16, 16
NW = NC * NS
CH = 64
NCH0 = 96
NCH1 = 64
NCHROWS = NS * (NCH0 + NCH1)
EP = NCHROWS * CH
NCHPAD = NCHROWS + NCH0 + 1 - NCH1
EPPAD = 2640 * CH
ROWS = 10240
RPS = ROWS // NS
RB = 1000
GRID = N // RB
RB1 = 2000
GRID1 = N // RB1


def _pack_words(hh):
    lo = lax.bitcast_convert_type(hh[:, :NBH // 2].astype(jnp.bfloat16),
                                  jnp.uint16).astype(jnp.uint32)
    hi = lax.bitcast_convert_type(hh[:, NBH // 2:].astype(jnp.bfloat16),
                                  jnp.uint16).astype(jnp.uint32)
    return (lo | (hi << 16)).astype(jnp.int32)


def _dot3(x, w):
    f32 = jnp.float32
    xh = x.astype(jnp.bfloat16)
    xl = (x - xh.astype(f32)).astype(jnp.bfloat16)
    wh = w.astype(jnp.bfloat16)
    wl = (w - wh.astype(f32)).astype(jnp.bfloat16)

    def d(a, b):
        return lax.dot_general(a, b, (((1,), (1,)), ((), ())),
                               preferred_element_type=f32)

    return d(xh, wh) + d(xl, wh) + d(xh, wl)


def _mm_body(seq_ref, wfc_ref, bfc_ref, wgat_ref, asv_ref, adv_ref,
             h0_ref, h1_ref, h0w_ref, h1w_ref, as_ref, ad_ref, mx_ref):
    x = seq_ref[...]
    ret = _dot3(x, wfc_ref[...]) + bfc_ref[...]
    h = _dot3(ret, wgat_ref[...])
    a_s = jnp.sum(h * asv_ref[...], axis=1)
    a_d = jnp.sum(h * adv_ref[...], axis=1)
    h0_ref[...] = h[:, :NBH]
    h1_ref[...] = h[:, NBH:]
    h0w_ref[...] = _pack_words(h[:, :NBH])
    h1w_ref[...] = _pack_words(h[:, NBH:])
    as_ref[...] = a_s[:, None]
    ad_ref[...] = a_d[:, None]
    bm = jnp.stack([jnp.max(a_s), jnp.max(a_d)])[None, :]

    @pl.when(pl.program_id(0) == 0)
    def _():
        mx_ref[...] = bm

    @pl.when(pl.program_id(0) > 0)
    def _():
        mx_ref[...] = jnp.maximum(mx_ref[...], bm)


def _sc_edge_kernel(src_hbm, dst_hbm, asrc_hbm, adst_hbm, m_hbm, h0_hbm,
                    h1_hbm, part_hbm, wpart_hbm,
                    src_v, dst_v, m_v, w_v,
                    feat_acc, w_acc, gsem0, gsem1):
    cid = lax.axis_index("c")
    sid = lax.axis_index("s")
    nchunk = jnp.where(cid == 0, NCH0, NCH1)
    row0 = jnp.where(cid == 0, sid * NCH0, NS * NCH0 + sid * NCH1)

    pltpu.sync_copy(src_hbm.at[pl.ds(row0, NCH0 + 1)], src_v)
    pltpu.sync_copy(dst_hbm.at[pl.ds(row0, NCH0)], dst_v)
    pltpu.sync_copy(m_hbm, m_v)

    m16 = m_v[...]
    iota = lax.iota(jnp.int32, LANES)

    def _weights(atab):
        pltpu.sync_copy(asrc_hbm, atab)

        @pl.loop(0, nchunk)
        def _(j):
            @pl.loop(0, CH // LANES)
            def _(k):
                s16 = src_v[j, pl.ds(k * LANES, LANES)]
                w_v[pl.ds(j * CH + k * LANES, LANES)] = plsc.load_gather(
                    atab, [s16])

        pltpu.sync_copy(adst_hbm, atab)

        @pl.loop(0, nchunk)
        def _(j):
            @pl.loop(0, CH // LANES)
            def _(k):
                d16 = dst_v[j, pl.ds(k * LANES, LANES)]
                u = w_v[pl.ds(j * CH + k * LANES, LANES)] + plsc.load_gather(
                    atab, [d16])
                e = jnp.where(u >= 0.0, u, 0.2 * u)
                w_v[pl.ds(j * CH + k * LANES, LANES)] = jnp.exp(e - m16)

    pl.run_scoped(_weights, pltpu.VMEM((ROWS,), jnp.float32))

    def _passes(rb0, rb1, stage, wstage):
        _feature_passes(src_v, dst_v, w_v, h0_hbm, h1_hbm, part_hbm,
                        wpart_hbm, feat_acc, w_acc, gsem0, gsem1, cid, sid,
                        iota, nchunk, rb0, rb1, stage, wstage)

    pl.run_scoped(_passes,
                  pltpu.VMEM((CH, NBH // 2), jnp.int32),
                  pltpu.VMEM((CH, NBH // 2), jnp.int32),
                  pltpu.VMEM((CH, NBH), jnp.float32),
                  pltpu.VMEM((CH, LANES), jnp.float32))


def _feature_passes(src_v, dst_v, w_v, h0_hbm, h1_hbm, part_hbm, wpart_hbm,
                    feat_acc, w_acc, gsem0, gsem1, cid, sid, iota, nchunk,
                    rb0, rb1, stage, wstage):
    def _scale(rb, j, f):
        @plsc.parallel_loop(0, CH, unroll=4)
        def _(i):
            idx16 = iota * 0 + (j * CH + i)
            wb = plsc.load_gather(w_v, [idx16])
            for c in range(NBH // (2 * LANES)):
                vi = rb[i, pl.ds(c * LANES, LANES)]
                lo = plsc.bitcast(vi << 16, jnp.float32)
                hi = plsc.bitcast(vi & jnp.int32(-65536), jnp.float32)
                stage[i, pl.ds(c * LANES, LANES)] = lo * wb
                stage[i, pl.ds(NBH // 2 + c * LANES, LANES)] = hi * wb
            if f == 0:
                wstage[i, pl.ds(0, LANES)] = jnp.where(iota == 0, wb, 0.0)

    for f in range(2):
        h_hbm = h0_hbm if f == 0 else h1_hbm

        @pl.loop(0, CH)
        def _(i):
            for c in range(NBH // LANES):
                stage[i, pl.ds(c * LANES, LANES)] = jnp.zeros(
                    (LANES,), jnp.float32)
            if f == 0:
                wstage[i, pl.ds(0, LANES)] = jnp.zeros((LANES,), jnp.float32)

        @pl.loop(0, RPS // CH)
        def _(r):
            off = sid * RPS + r * CH
            pltpu.sync_copy(stage, feat_acc.at[pl.ds(off, CH)])
            if f == 0:
                pltpu.sync_copy(wstage, w_acc.at[pl.ds(off, CH)])

        plsc.subcore_barrier()

        pltpu.make_async_copy(h_hbm.at[src_v.at[0]], rb0, gsem0).start()

        @pl.loop(0, nchunk // 2)
        def _(t):
            c0 = 2 * t
            c1 = c0 + 1
            pltpu.make_async_copy(h_hbm.at[src_v.at[c1]], rb1, gsem1).start()
            pltpu.make_async_copy(h_hbm.at[src_v.at[c0]], rb0, gsem0).wait()
            _scale(rb0, c0, f)
            pltpu.sync_copy(stage, feat_acc.at[dst_v.at[c0]], add=True)
            if f == 0:
                pltpu.sync_copy(wstage, w_acc.at[dst_v.at[c0]], add=True)
            pltpu.make_async_copy(h_hbm.at[src_v.at[c0 + 2]], rb0,
                                  gsem0).start()
            pltpu.make_async_copy(h_hbm.at[src_v.at[c1]], rb1, gsem1).wait()
            _scale(rb1, c1, f)
            pltpu.sync_copy(stage, feat_acc.at[dst_v.at[c1]], add=True)
            if f == 0:
                pltpu.sync_copy(wstage, w_acc.at[dst_v.at[c1]], add=True)

        pltpu.make_async_copy(h_hbm.at[src_v.at[nchunk]], rb0, gsem0).wait()

        plsc.subcore_barrier()

        @pl.loop(0, RPS // CH)
        def _(r):
            off = sid * RPS + r * CH
            pltpu.sync_copy(feat_acc.at[pl.ds(off, CH)],
                            part_hbm.at[cid, f, pl.ds(off, CH)])
            if f == 0:
                pltpu.sync_copy(w_acc.at[pl.ds(off, CH)],
                                wpart_hbm.at[cid, pl.ds(off, CH)])

        plsc.subcore_barrier()


def _combine_body(part_ref, wpart_ref, h0_ref, h1_ref, as_ref, ad_ref, m_ref,
                  b_ref, p_ref, out_ref):
    P = part_ref[...]
    W = wpart_ref[:, pl.ds(pl.program_id(0) * (RB // 8), RB // 8), :]
    num0 = P[0, 0] + P[1, 0]
    num1 = P[0, 1] + P[1, 1]
    wsum = (W[0] + W[1]).reshape(RB // 8, 8, LANES)[:, :, 0].reshape(RB)
    M = m_ref[0, 0] + m_ref[0, 1]
    u = as_ref[...][:, 0] + ad_ref[...][:, 0]
    e = jnp.where(u >= 0.0, u, 0.2 * u)
    wself = jnp.exp(e - M)
    denom = wsum + wself
    o0 = (num0 + wself[:, None] * h0_ref[...]) / denom[:, None]
    o1 = (num1 + wself[:, None] * h1_ref[...]) / denom[:, None]
    o = jnp.concatenate([o0, o1], axis=1) + b_ref[...]
    a = p_ref[0, 0]
    out_ref[...] = jnp.where(o >= 0.0, o, a * o)


def kernel(seq, edge_index, W_fc, b_fc, W_gat, att_src, att_dst, bias_gat,
           prelu_a):
    f32 = jnp.float32

    h0, h1, h0w, h1w, as2, ad2, mx = pl.pallas_call(
        _mm_body,
        grid=(GRID1,),
        in_specs=[
            pl.BlockSpec((RB1, FT_IN), lambda i: (i, 0)),
            pl.BlockSpec((NB, FT_IN), lambda i: (0, 0)),
            pl.BlockSpec((1, NB), lambda i: (0, 0)),
            pl.BlockSpec((NB, NB), lambda i: (0, 0)),
            pl.BlockSpec((1, NB), lambda i: (0, 0)),
            pl.BlockSpec((1, NB), lambda i: (0, 0)),
        ],
        out_specs=[
            pl.BlockSpec((RB1, NBH), lambda i: (i, 0)),
            pl.BlockSpec((RB1, NBH), lambda i: (i, 0)),
            pl.BlockSpec((RB1, NBH // 2), lambda i: (i, 0)),
            pl.BlockSpec((RB1, NBH // 2), lambda i: (i, 0)),
            pl.BlockSpec((RB1, 1), lambda i: (i, 0)),
            pl.BlockSpec((RB1, 1), lambda i: (i, 0)),
            pl.BlockSpec((1, 2), lambda i: (0, 0)),
        ],
        out_shape=[
            jax.ShapeDtypeStruct((N, NBH), f32),
            jax.ShapeDtypeStruct((N, NBH), f32),
            jax.ShapeDtypeStruct((N, NBH // 2), jnp.int32),
            jax.ShapeDtypeStruct((N, NBH // 2), jnp.int32),
            jax.ShapeDtypeStruct((ROWS, 1), f32),
            jax.ShapeDtypeStruct((ROWS, 1), f32),
            jax.ShapeDtypeStruct((1, 2), f32),
        ],
    )(seq, W_fc, b_fc.reshape(1, NB), W_gat, att_src.reshape(1, NB),
      att_dst.reshape(1, NB))

    src = jnp.concatenate(
        [edge_index[0], jnp.zeros((EPPAD - E,), jnp.int32)]).reshape(
            EPPAD // CH, CH)
    dst = jnp.concatenate(
        [edge_index[1], jnp.full((EPPAD - E,), N, jnp.int32)]).reshape(
            EPPAD // CH, CH)
    asrc_p = as2.reshape(ROWS)
    adst_p = ad2.reshape(ROWS)
    m16 = jnp.full((LANES,), mx[0, 0] + mx[0, 1], f32)

    mesh = plsc.VectorSubcoreMesh(core_axis_name="c", subcore_axis_name="s")
    cp = pltpu.CompilerParams(needs_layout_passes=False,
                              use_tc_tiling_on_sc=False)
    sc_kernel = pl.kernel(
        _sc_edge_kernel,
        out_type=[
            jax.ShapeDtypeStruct((NC, 2, ROWS, NBH), f32),
            jax.ShapeDtypeStruct((NC, ROWS, LANES), f32),
        ],
        mesh=mesh,
        compiler_params=cp,
        scratch_types=[
            pltpu.VMEM((NCH0 + 1, CH), jnp.int32),
            pltpu.VMEM((NCH0, CH), jnp.int32),
            pltpu.VMEM((LANES,), f32),
            pltpu.VMEM((NCH0 * CH,), f32),
            pltpu.VMEM_SHARED((ROWS, NBH), f32),
            pltpu.VMEM_SHARED((ROWS, LANES), f32),
            pltpu.SemaphoreType.DMA,
            pltpu.SemaphoreType.DMA,
        ],
    )
    part, wpart = sc_kernel(src, dst, asrc_p, adst_p, m16, h0w, h1w)

    out = pl.pallas_call(
        _combine_body,
        grid=(GRID,),
        in_specs=[
            pl.BlockSpec((NC, 2, RB, NBH), lambda i: (0, 0, i, 0)),
            pl.BlockSpec((NC, ROWS // 8, 8 * LANES), lambda i: (0, 0, 0)),
            pl.BlockSpec((RB, NBH), lambda i: (i, 0)),
            pl.BlockSpec((RB, NBH), lambda i: (i, 0)),
            pl.BlockSpec((RB, 1), lambda i: (i, 0)),
            pl.BlockSpec((RB, 1), lambda i: (i, 0)),
            pl.BlockSpec((1, 2), lambda i: (0, 0)),
            pl.BlockSpec((1, NB), lambda i: (0, 0)),
            pl.BlockSpec((1, 1), lambda i: (0, 0)),
        ],
        out_specs=pl.BlockSpec((RB, NB), lambda i: (i, 0)),
        out_shape=jax.ShapeDtypeStruct((N, NB), f32),
    )(part, wpart.reshape(NC, ROWS // 8, 8 * LANES), h0, h1, as2, ad2, mx,
      bias_gat.reshape(1, NB),
      prelu_a.reshape(1, 1))
    return out


# speedup vs baseline: 15.8087x; 1.0021x over previous
import jax
import jax.numpy as jnp
from jax import lax
from jax.experimental import pallas as pl
from jax.experimental.pallas import tpu as pltpu
from jax.experimental.pallas import tpu_sc as plsc

N = 10000
E = 160000
FT_IN = 512
NB = 256
NBH = NB // 2
NC, NS, LANES = 2, 16, 16
NW = NC * NS
CH = 64
NCH0 = 96
NCH1 = 64
NCHROWS = NS * (NCH0 + NCH1)
EP = NCHROWS * CH
EPPAD = 2640 * CH
ROWS = 10240
RPS = ROWS // NS
RB = 1000
GRID = N // RB
RB1 = 2000
GRID1 = N // RB1


def _pack_words(hh):
    lo = lax.bitcast_convert_type(hh[:, :NBH // 2].astype(jnp.bfloat16),
                                  jnp.uint16).astype(jnp.uint32)
    hi = lax.bitcast_convert_type(hh[:, NBH // 2:].astype(jnp.bfloat16),
                                  jnp.uint16).astype(jnp.uint32)
    return (lo | (hi << 16)).astype(jnp.int32)


def _dot3(x, w):
    f32 = jnp.float32
    xh = x.astype(jnp.bfloat16)
    xl = (x - xh.astype(f32)).astype(jnp.bfloat16)
    wh = w.astype(jnp.bfloat16)
    wl = (w - wh.astype(f32)).astype(jnp.bfloat16)

    def d(a, b):
        return lax.dot_general(a, b, (((1,), (1,)), ((), ())),
                               preferred_element_type=f32)

    return d(xh, wh) + d(xl, wh) + d(xh, wl)


def _mm_body(seq_ref, wfc_ref, bfc_ref, wgat_ref, asv_ref, adv_ref,
             h0_ref, h1_ref, h0w_ref, h1w_ref, as_ref, ad_ref, mx_ref):
    x = seq_ref[...]
    ret = _dot3(x, wfc_ref[...]) + bfc_ref[...]
    h = _dot3(ret, wgat_ref[...])
    a_s = jnp.sum(h * asv_ref[...], axis=1)
    a_d = jnp.sum(h * adv_ref[...], axis=1)
    h0_ref[...] = h[:, :NBH]
    h1_ref[...] = h[:, NBH:]
    h0w_ref[...] = _pack_words(h[:, :NBH])
    h1w_ref[...] = _pack_words(h[:, NBH:])
    as_ref[...] = a_s[:, None]
    ad_ref[...] = a_d[:, None]
    bm = jnp.stack([jnp.max(a_s), jnp.max(a_d)])[None, :]

    @pl.when(pl.program_id(0) == 0)
    def _():
        mx_ref[...] = bm

    @pl.when(pl.program_id(0) > 0)
    def _():
        mx_ref[...] = jnp.maximum(mx_ref[...], bm)


def _sc_edge_kernel(src_hbm, dst_hbm, asrc_hbm, adst_hbm, m_hbm, h0_hbm,
                    h1_hbm, part_hbm, wpart_hbm,
                    src_v, dst_v, m_v, w_v,
                    feat_acc, w_acc, gsem0, gsem1):
    cid = lax.axis_index("c")
    sid = lax.axis_index("s")
    nchunk = jnp.where(cid == 0, NCH0, NCH1)
    row0 = jnp.where(cid == 0, sid * NCH0, NS * NCH0 + sid * NCH1)

    pltpu.sync_copy(src_hbm.at[pl.ds(row0, NCH0 + 1)], src_v)
    pltpu.sync_copy(dst_hbm.at[pl.ds(row0, NCH0)], dst_v)
    pltpu.sync_copy(m_hbm, m_v)

    m16 = m_v[...]
    iota = lax.iota(jnp.int32, LANES)

    def _weights(atab):
        pltpu.sync_copy(asrc_hbm, atab)

        @pl.loop(0, nchunk)
        def _(j):
            @pl.loop(0, CH // LANES)
            def _(k):
                s16 = src_v[j, pl.ds(k * LANES, LANES)]
                w_v[pl.ds(j * CH + k * LANES, LANES)] = plsc.load_gather(
                    atab, [s16])

        pltpu.sync_copy(adst_hbm, atab)

        @pl.loop(0, nchunk)
        def _(j):
            @pl.loop(0, CH // LANES)
            def _(k):
                d16 = dst_v[j, pl.ds(k * LANES, LANES)]
                u = w_v[pl.ds(j * CH + k * LANES, LANES)] + plsc.load_gather(
                    atab, [d16])
                e = jnp.where(u >= 0.0, u, 0.2 * u)
                w_v[pl.ds(j * CH + k * LANES, LANES)] = jnp.exp(e - m16)

    pl.run_scoped(_weights, pltpu.VMEM((ROWS,), jnp.float32))

    def _passes(rb0, rb1, stage, wstage):
        _feature_passes(src_v, dst_v, w_v, h0_hbm, h1_hbm, part_hbm,
                        wpart_hbm, feat_acc, w_acc, gsem0, gsem1, cid, sid,
                        iota, nchunk, rb0, rb1, stage, wstage)

    pl.run_scoped(_passes,
                  pltpu.VMEM((CH, NBH // 2), jnp.int32),
                  pltpu.VMEM((CH, NBH // 2), jnp.int32),
                  pltpu.VMEM((CH, NBH), jnp.float32),
                  pltpu.VMEM((CH, LANES), jnp.float32))


def _feature_passes(src_v, dst_v, w_v, h0_hbm, h1_hbm, part_hbm, wpart_hbm,
                    feat_acc, w_acc, gsem0, gsem1, cid, sid, iota, nchunk,
                    rb0, rb1, stage, wstage):
    def _scale(rb, j, f):
        @plsc.parallel_loop(0, CH, unroll=4)
        def _(i):
            idx16 = iota * 0 + (j * CH + i)
            wb = plsc.load_gather(w_v, [idx16])
            for c in range(NBH // (2 * LANES)):
                vi = rb[i, pl.ds(c * LANES, LANES)]
                lo = plsc.bitcast(vi << 16, jnp.float32)
                hi = plsc.bitcast(vi & jnp.int32(-65536), jnp.float32)
                stage[i, pl.ds(c * LANES, LANES)] = lo * wb
                stage[i, pl.ds(NBH // 2 + c * LANES, LANES)] = hi * wb
            if f == 0:
                wstage[i, pl.ds(0, LANES)] = jnp.where(iota == 0, wb, 0.0)

    for f in range(2):
        h_hbm = h0_hbm if f == 0 else h1_hbm

        @pl.loop(0, CH)
        def _(i):
            for c in range(NBH // LANES):
                stage[i, pl.ds(c * LANES, LANES)] = jnp.zeros(
                    (LANES,), jnp.float32)
            if f == 0:
                wstage[i, pl.ds(0, LANES)] = jnp.zeros((LANES,), jnp.float32)

        @pl.loop(0, RPS // CH)
        def _(r):
            off = sid * RPS + r * CH
            pltpu.sync_copy(stage, feat_acc.at[pl.ds(off, CH)])
            if f == 0:
                pltpu.sync_copy(wstage, w_acc.at[pl.ds(off, CH)])

        plsc.subcore_barrier()

        pltpu.make_async_copy(h_hbm.at[src_v.at[0]], rb0, gsem0).start()

        @pl.loop(0, nchunk // 2)
        def _(t):
            c0 = 2 * t
            c1 = c0 + 1
            pltpu.make_async_copy(h_hbm.at[src_v.at[c1]], rb1, gsem1).start()
            pltpu.make_async_copy(h_hbm.at[src_v.at[c0]], rb0, gsem0).wait()
            _scale(rb0, c0, f)
            pltpu.sync_copy(stage, feat_acc.at[dst_v.at[c0]], add=True)
            if f == 0:
                pltpu.sync_copy(wstage, w_acc.at[dst_v.at[c0]], add=True)
            pltpu.make_async_copy(h_hbm.at[src_v.at[c0 + 2]], rb0,
                                  gsem0).start()
            pltpu.make_async_copy(h_hbm.at[src_v.at[c1]], rb1, gsem1).wait()
            _scale(rb1, c1, f)
            pltpu.sync_copy(stage, feat_acc.at[dst_v.at[c1]], add=True)
            if f == 0:
                pltpu.sync_copy(wstage, w_acc.at[dst_v.at[c1]], add=True)

        pltpu.make_async_copy(h_hbm.at[src_v.at[nchunk]], rb0, gsem0).wait()

        plsc.subcore_barrier()

        @pl.loop(0, RPS // CH)
        def _(r):
            off = sid * RPS + r * CH
            pltpu.sync_copy(feat_acc.at[pl.ds(off, CH)],
                            part_hbm.at[cid, f, pl.ds(off, CH)])
            if f == 0:
                pltpu.sync_copy(w_acc.at[pl.ds(off, CH)],
                                wpart_hbm.at[cid, pl.ds(off, CH)])

        plsc.subcore_barrier()


def _combine_body(part_ref, wpart_ref, h0_ref, h1_ref, as_ref, ad_ref, m_ref,
                  b_ref, p_ref, out_ref):
    P = part_ref[...]
    W = wpart_ref[:, pl.ds(pl.program_id(0) * (RB // 8), RB // 8), :]
    num0 = P[0, 0] + P[1, 0]
    num1 = P[0, 1] + P[1, 1]
    wsum = (W[0] + W[1]).reshape(RB // 8, 8, LANES)[:, :, 0].reshape(RB)
    M = m_ref[0, 0] + m_ref[0, 1]
    u = as_ref[...][:, 0] + ad_ref[...][:, 0]
    e = jnp.where(u >= 0.0, u, 0.2 * u)
    wself = jnp.exp(e - M)
    denom = wsum + wself
    o0 = (num0 + wself[:, None] * h0_ref[...]) / denom[:, None]
    o1 = (num1 + wself[:, None] * h1_ref[...]) / denom[:, None]
    o = jnp.concatenate([o0, o1], axis=1) + b_ref[...]
    a = p_ref[0, 0]
    out_ref[...] = jnp.where(o >= 0.0, o, a * o)


def kernel(seq, edge_index, W_fc, b_fc, W_gat, att_src, att_dst, bias_gat,
           prelu_a):
    f32 = jnp.float32

    h0, h1, h0w, h1w, as2, ad2, mx = pl.pallas_call(
        _mm_body,
        grid=(GRID1,),
        in_specs=[
            pl.BlockSpec((RB1, FT_IN), lambda i: (i, 0)),
            pl.BlockSpec((NB, FT_IN), lambda i: (0, 0)),
            pl.BlockSpec((1, NB), lambda i: (0, 0)),
            pl.BlockSpec((NB, NB), lambda i: (0, 0)),
            pl.BlockSpec((1, NB), lambda i: (0, 0)),
            pl.BlockSpec((1, NB), lambda i: (0, 0)),
        ],
        out_specs=[
            pl.BlockSpec((RB1, NBH), lambda i: (i, 0)),
            pl.BlockSpec((RB1, NBH), lambda i: (i, 0)),
            pl.BlockSpec((RB1, NBH // 2), lambda i: (i, 0)),
            pl.BlockSpec((RB1, NBH // 2), lambda i: (i, 0)),
            pl.BlockSpec((RB1, 1), lambda i: (i, 0)),
            pl.BlockSpec((RB1, 1), lambda i: (i, 0)),
            pl.BlockSpec((1, 2), lambda i: (0, 0)),
        ],
        out_shape=[
            jax.ShapeDtypeStruct((N, NBH), f32),
            jax.ShapeDtypeStruct((N, NBH), f32),
            jax.ShapeDtypeStruct((N, NBH // 2), jnp.int32),
            jax.ShapeDtypeStruct((N, NBH // 2), jnp.int32),
            jax.ShapeDtypeStruct((ROWS, 1), f32),
            jax.ShapeDtypeStruct((ROWS, 1), f32),
            jax.ShapeDtypeStruct((1, 2), f32),
        ],
    )(seq, W_fc, b_fc.reshape(1, NB), W_gat, att_src.reshape(1, NB),
      att_dst.reshape(1, NB))

    src = jnp.concatenate(
        [edge_index[0], jnp.zeros((EPPAD - E,), jnp.int32)]).reshape(
            EPPAD // CH, CH)
    dst = jnp.concatenate(
        [edge_index[1], jnp.full((EPPAD - E,), N, jnp.int32)]).reshape(
            EPPAD // CH, CH)
    asrc_p = as2.reshape(ROWS)
    adst_p = ad2.reshape(ROWS)
    m16 = jnp.full((LANES,), mx[0, 0] + mx[0, 1], f32)

    mesh = plsc.VectorSubcoreMesh(core_axis_name="c", subcore_axis_name="s")
    cp = pltpu.CompilerParams(needs_layout_passes=False,
                              use_tc_tiling_on_sc=False)
    sc_kernel = pl.kernel(
        _sc_edge_kernel,
        out_type=[
            jax.ShapeDtypeStruct((NC, 2, ROWS, NBH), f32),
            jax.ShapeDtypeStruct((NC, ROWS, LANES), f32),
        ],
        mesh=mesh,
        compiler_params=cp,
        scratch_types=[
            pltpu.VMEM((NCH0 + 1, CH), jnp.int32),
            pltpu.VMEM((NCH0, CH), jnp.int32),
            pltpu.VMEM((LANES,), f32),
            pltpu.VMEM((NCH0 * CH,), f32),
            pltpu.VMEM_SHARED((ROWS, NBH), f32),
            pltpu.VMEM_SHARED((ROWS, LANES), f32),
            pltpu.SemaphoreType.DMA,
            pltpu.SemaphoreType.DMA,
        ],
    )
    part, wpart = sc_kernel(src, dst, asrc_p, adst_p, m16, h0w, h1w)

    out = pl.pallas_call(
        _combine_body,
        grid=(GRID,),
        in_specs=[
            pl.BlockSpec((NC, 2, RB, NBH), lambda i: (0, 0, i, 0)),
            pl.BlockSpec((NC, ROWS // 8, 8 * LANES), lambda i: (0, 0, 0)),
            pl.BlockSpec((RB, NBH), lambda i: (i, 0)),
            pl.BlockSpec((RB, NBH), lambda i: (i, 0)),
            pl.BlockSpec((RB, 1), lambda i: (i, 0)),
            pl.BlockSpec((RB, 1), lambda i: (i, 0)),
            pl.BlockSpec((1, 2), lambda i: (0, 0)),
            pl.BlockSpec((1, NB), lambda i: (0, 0)),
            pl.BlockSpec((1, 1), lambda i: (0, 0)),
        ],
        out_specs=pl.BlockSpec((RB, NB), lambda i: (i, 0)),
        out_shape=jax.ShapeDtypeStruct((N, NB), f32),
    )(part, wpart.reshape(NC, ROWS // 8, 8 * LANES), h0, h1, as2, ad2, mx,
      bias_gat.reshape(1, NB),
      prelu_a.reshape(1, 1))
    return out
